# baseline (speedup 1.0000x reference)
import functools

import jax
import jax.numpy as jnp
from jax import lax
from jax.experimental import pallas as pl
from jax.experimental.pallas import tpu as pltpu

N = 3072
D = 256
SH = 64
E = 98304


def _leaky(x):
    return jnp.where(x >= 0, x, 0.01 * x)


def _edge_prep_body(feat_ref, wpT_ref, w1aT_ref, w1bT_ref, b1_ref, a_ref, b_ref):
    p = jnp.dot(feat_ref[...], wpT_ref[...], preferred_element_type=jnp.float32)
    nrm = jnp.sqrt(jnp.sum(p * p, axis=1, keepdims=True))
    pn = p / jnp.maximum(nrm, 1e-12)
    a_ref[...] = jnp.dot(pn, w1aT_ref[...], preferred_element_type=jnp.float32) + b1_ref[...]
    b_ref[...] = jnp.dot(pn, w1bT_ref[...], preferred_element_type=jnp.float32)


def _edge_prep(feat, wpT, w1aT, w1bT, b1, *, interpret=False):
    BM = 512
    grid = (N // BM,)
    return pl.pallas_call(
        _edge_prep_body,
        grid=grid,
        in_specs=[
            pl.BlockSpec((BM, D), lambda i: (i, 0)),
            pl.BlockSpec((D, SH), lambda i: (0, 0)),
            pl.BlockSpec((SH, SH), lambda i: (0, 0)),
            pl.BlockSpec((SH, SH), lambda i: (0, 0)),
            pl.BlockSpec((1, SH), lambda i: (0, 0)),
        ],
        out_specs=[
            pl.BlockSpec((BM, SH), lambda i: (i, 0)),
            pl.BlockSpec((BM, SH), lambda i: (i, 0)),
        ],
        out_shape=[
            jax.ShapeDtypeStruct((N, SH), jnp.float32),
            jax.ShapeDtypeStruct((N, SH), jnp.float32),
        ],
        interpret=interpret,
    )(feat, wpT, w1aT, w1bT, b1)


def _edge_mlp_body(h_ref, w2_ref, b2_ref, w_ref):
    l = _leaky(h_ref[...])
    s = jnp.sum(l * w2_ref[...], axis=1) + b2_ref[0]
    w_ref[...] = jax.nn.sigmoid(s)


def _edge_mlp(h, w2row, b2, *, interpret=False):
    BE = 12288
    grid = (E // BE,)
    return pl.pallas_call(
        _edge_mlp_body,
        grid=grid,
        in_specs=[
            pl.BlockSpec((BE, SH), lambda i: (i, 0)),
            pl.BlockSpec((1, SH), lambda i: (0, 0)),
            pl.BlockSpec(memory_space=pltpu.SMEM),
        ],
        out_specs=pl.BlockSpec((BE,), lambda i: (i,)),
        out_shape=jax.ShapeDtypeStruct((E,), jnp.float32),
        interpret=interpret,
    )(h, w2row, b2)


def _q_body(x_ref, w1T_ref, rs_ref, q_ref):
    i = pl.program_id(0)
    rs = rs_ref[pl.ds(i * 512, 512), :]
    d = jnp.where(rs > 0, lax.rsqrt(jnp.where(rs > 0, rs, 1.0)), 0.0)
    q_ref[...] = jnp.dot(x_ref[...], w1T_ref[...], preferred_element_type=jnp.float32) * d


def _q_step(x, w1T, rs2d, *, interpret=False):
    BM = 512
    grid = (N // BM,)
    return pl.pallas_call(
        _q_body,
        grid=grid,
        in_specs=[
            pl.BlockSpec((BM, D), lambda i: (i, 0)),
            pl.BlockSpec((D, D), lambda i: (0, 0)),
            pl.BlockSpec((N, 1), lambda i: (0, 0)),
        ],
        out_specs=pl.BlockSpec((BM, D), lambda i: (i, 0)),
        out_shape=jax.ShapeDtypeStruct((N, D), jnp.float32),
        interpret=interpret,
    )(x, w1T, rs2d)


def _layer_body(adj_ref, q_ref, x_ref, rs_ref, w2aT_ref, w2bT_ref, out_ref, acc_ref):
    m = pl.program_id(0)
    k = pl.program_id(1)
    nk = pl.num_programs(1)
    part = jnp.dot(adj_ref[...], q_ref[...], preferred_element_type=jnp.float32)

    @pl.when(k == 0)
    def _():
        acc_ref[...] = part

    @pl.when(k != 0)
    def _():
        acc_ref[...] += part

    @pl.when(k == nk - 1)
    def _():
        rs = rs_ref[pl.ds(m * 512, 512), :]
        d = jnp.where(rs > 0, lax.rsqrt(jnp.where(rs > 0, rs, 1.0)), 0.0)
        nb = acc_ref[...] * d
        x = x_ref[...]
        u = x + nb
        v = x * nb
        out_ref[...] = _leaky(
            jnp.dot(u, w2aT_ref[...], preferred_element_type=jnp.float32)
            + jnp.dot(v, w2bT_ref[...], preferred_element_type=jnp.float32))


def _layer_step(adj, q, x, rs2d, w2aT, w2bT, *, interpret=False):
    BM, BK = 512, 512
    grid = (N // BM, N // BK)
    return pl.pallas_call(
        _layer_body,
        grid=grid,
        in_specs=[
            pl.BlockSpec((BM, BK), lambda m, k: (m, k)),
            pl.BlockSpec((BK, D), lambda m, k: (k, 0)),
            pl.BlockSpec((BM, D), lambda m, k: (m, 0)),
            pl.BlockSpec((N, 1), lambda m, k: (0, 0)),
            pl.BlockSpec((D, D), lambda m, k: (0, 0)),
            pl.BlockSpec((D, D), lambda m, k: (0, 0)),
        ],
        out_specs=pl.BlockSpec((BM, D), lambda m, k: (m, 0)),
        out_shape=jax.ShapeDtypeStruct((N, D), jnp.float32),
        scratch_shapes=[pltpu.VMEM((BM, D), jnp.float32)],
        interpret=interpret,
    )(adj, q, x, rs2d, w2aT, w2bT)


def kernel(feature_tuple, dia_lens, win_p, win_f, edge_index, ewg_proj_W,
           mlp_W1, mlp_b1, mlp_W2, mlp_b2, l0_W1, l0_W2, l1_W1, l1_W2,
           *, interpret=False):
    feat = feature_tuple.reshape(-1, feature_tuple.shape[-1])
    src, tgt = edge_index[0], edge_index[1]

    wpT = ewg_proj_W.T
    w1aT = mlp_W1[:, :SH].T
    w1bT = mlp_W1[:, SH:].T
    b1 = mlp_b1.reshape(1, SH)
    a_tab, b_tab = _edge_prep(feat, wpT, w1aT, w1bT, b1, interpret=interpret)

    h = a_tab[src] + b_tab[tgt]

    w = _edge_mlp(h, mlp_W2.reshape(1, SH), mlp_b2, interpret=interpret)

    adjv = jnp.zeros((N * N,), jnp.float32).at[src * N + tgt].add(w)
    adj = adjv.reshape(N, N)
    rs = jnp.zeros((N,), jnp.float32).at[src].add(w)
    rs2d = rs.reshape(N, 1)

    x = feat
    for (W1, W2) in [(l0_W1, l0_W2), (l1_W1, l1_W2)]:
        q = _q_step(x, W1.T, rs2d, interpret=interpret)
        x = _layer_step(adj, q, x, rs2d, W2[:, :D].T, W2[:, D:].T,
                        interpret=interpret)

    f0, f1, f2 = jnp.split(x, 3, axis=0)
    return (f0, f1, f2, edge_index)


# speedup vs baseline: 3.0728x; 3.0728x over previous
import functools

import jax
import jax.numpy as jnp
from jax import lax
from jax.experimental import pallas as pl
from jax.experimental.pallas import tpu as pltpu
from jax.experimental.pallas import tpu_sc as plsc

N = 3072
D = 256
SH = 64
E = 98304


def _leaky(x):
    return jnp.where(x >= 0, x, 0.01 * x)


def _edge_prep_body(feat_ref, wpT_ref, w1aT_ref, w1bT_ref, b1_ref, t_ref):
    p = jnp.dot(feat_ref[...], wpT_ref[...], preferred_element_type=jnp.float32)
    nrm = jnp.sqrt(jnp.sum(p * p, axis=1, keepdims=True))
    pn = p / jnp.maximum(nrm, 1e-12)
    a = jnp.dot(pn, w1aT_ref[...], preferred_element_type=jnp.float32) + b1_ref[...]
    b = jnp.dot(pn, w1bT_ref[...], preferred_element_type=jnp.float32)
    t_ref[...] = jnp.concatenate([a, b], axis=-1)


def _edge_prep(feat, wpT, w1aT, w1bT, b1, *, interpret=False):
    BM = 512
    grid = (N // BM,)
    return pl.pallas_call(
        _edge_prep_body,
        grid=grid,
        in_specs=[
            pl.BlockSpec((BM, D), lambda i: (i, 0)),
            pl.BlockSpec((D, SH), lambda i: (0, 0)),
            pl.BlockSpec((SH, SH), lambda i: (0, 0)),
            pl.BlockSpec((SH, SH), lambda i: (0, 0)),
            pl.BlockSpec((1, SH), lambda i: (0, 0)),
        ],
        out_specs=pl.BlockSpec((BM, 2 * SH), lambda i: (i, 0)),
        out_shape=jax.ShapeDtypeStruct((N, 2 * SH), jnp.float32),
        interpret=interpret,
    )(feat, wpT, w1aT, w1bT, b1)


def _edge_mlp_body(g1_ref, g2_ref, w2_ref, b2_ref, w_ref):
    l = _leaky(g1_ref[:, :SH] + g2_ref[:, SH:])
    s = jnp.sum(l * w2_ref[...], axis=1) + b2_ref[0]
    w_ref[...] = jax.nn.sigmoid(s)


def _edge_mlp(g1, g2, w2row, b2, *, interpret=False):
    BE = 12288
    grid = (E // BE,)
    return pl.pallas_call(
        _edge_mlp_body,
        grid=grid,
        in_specs=[
            pl.BlockSpec((BE, 2 * SH), lambda i: (i, 0)),
            pl.BlockSpec((BE, 2 * SH), lambda i: (i, 0)),
            pl.BlockSpec((1, SH), lambda i: (0, 0)),
            pl.BlockSpec(memory_space=pltpu.SMEM),
        ],
        out_specs=pl.BlockSpec((BE,), lambda i: (i,)),
        out_shape=jax.ShapeDtypeStruct((E,), jnp.float32),
        interpret=interpret,
    )(g1, g2, w2row, b2)


def _q_body(x_ref, w1T_ref, rs_ref, q_ref):
    i = pl.program_id(0)
    rs = rs_ref[pl.ds(i * 512, 512), :]
    d = jnp.where(rs > 0, lax.rsqrt(jnp.where(rs > 0, rs, 1.0)), 0.0)
    q_ref[...] = jnp.dot(x_ref[...], w1T_ref[...], preferred_element_type=jnp.float32) * d


def _q_step(x, w1T, rs2d, *, interpret=False):
    BM = 512
    grid = (N // BM,)
    return pl.pallas_call(
        _q_body,
        grid=grid,
        in_specs=[
            pl.BlockSpec((BM, D), lambda i: (i, 0)),
            pl.BlockSpec((D, D), lambda i: (0, 0)),
            pl.BlockSpec((N, 1), lambda i: (0, 0)),
        ],
        out_specs=pl.BlockSpec((BM, D), lambda i: (i, 0)),
        out_shape=jax.ShapeDtypeStruct((N, D), jnp.float32),
        interpret=interpret,
    )(x, w1T, rs2d)


def _layer_body(adj_ref, q_ref, x_ref, rs_ref, w2aT_ref, w2bT_ref, out_ref, acc_ref):
    m = pl.program_id(0)
    k = pl.program_id(1)
    nk = pl.num_programs(1)
    part = jnp.dot(adj_ref[...], q_ref[...], preferred_element_type=jnp.float32)

    @pl.when(k == 0)
    def _():
        acc_ref[...] = part

    @pl.when(k != 0)
    def _():
        acc_ref[...] += part

    @pl.when(k == nk - 1)
    def _():
        rs = rs_ref[pl.ds(m * 512, 512), :]
        d = jnp.where(rs > 0, lax.rsqrt(jnp.where(rs > 0, rs, 1.0)), 0.0)
        nb = acc_ref[...] * d
        x = x_ref[...]
        u = x + nb
        v = x * nb
        out_ref[...] = _leaky(
            jnp.dot(u, w2aT_ref[...], preferred_element_type=jnp.float32)
            + jnp.dot(v, w2bT_ref[...], preferred_element_type=jnp.float32))


def _layer_step(adj, q, x, rs2d, w2aT, w2bT, *, interpret=False):
    BM, BK = 512, 512
    grid = (N // BM, N // BK)
    return pl.pallas_call(
        _layer_body,
        grid=grid,
        in_specs=[
            pl.BlockSpec((BM, BK), lambda m, k: (m, k)),
            pl.BlockSpec((BK, D), lambda m, k: (k, 0)),
            pl.BlockSpec((BM, D), lambda m, k: (m, 0)),
            pl.BlockSpec((N, 1), lambda m, k: (0, 0)),
            pl.BlockSpec((D, D), lambda m, k: (0, 0)),
            pl.BlockSpec((D, D), lambda m, k: (0, 0)),
        ],
        out_specs=pl.BlockSpec((BM, D), lambda m, k: (m, 0)),
        out_shape=jax.ShapeDtypeStruct((N, D), jnp.float32),
        scratch_shapes=[pltpu.VMEM((BM, D), jnp.float32)],
        interpret=interpret,
    )(adj, q, x, rs2d, w2aT, w2bT)


def _sc_gather(t_tab, src, tgt):
    EW = E // 32
    CH = 256
    NCH = EW // CH

    mesh = plsc.VectorSubcoreMesh(core_axis_name="c", subcore_axis_name="s")

    @functools.partial(
        pl.kernel,
        out_type=[jax.ShapeDtypeStruct((E, 2 * SH), jnp.float32),
                  jax.ShapeDtypeStruct((E, 2 * SH), jnp.float32)],
        mesh=mesh,
        scratch_types=[
            pltpu.VMEM((EW,), jnp.int32),
            pltpu.VMEM((EW,), jnp.int32),
            pltpu.VMEM((CH, 2 * SH), jnp.float32),
            pltpu.VMEM((CH, 2 * SH), jnp.float32),
            pltpu.SemaphoreType.DMA,
        ],
    )
    def k(t_hbm, src_hbm, tgt_hbm, g1_hbm, g2_hbm, idxs, idxt, ra, rb, sem):
        wid = lax.axis_index("s") * 2 + lax.axis_index("c")
        base = wid * EW
        pltpu.sync_copy(src_hbm.at[pl.ds(base, EW)], idxs)
        pltpu.sync_copy(tgt_hbm.at[pl.ds(base, EW)], idxt)

        def chunk(ci, carry):
            descs = []
            for j in range(CH // 128):
                off = ci * CH + j * 128
                descs.append(pltpu.async_copy(
                    t_hbm.at[idxs.at[pl.ds(off, 128)]],
                    ra.at[pl.ds(j * 128, 128)], sem))
                descs.append(pltpu.async_copy(
                    t_hbm.at[idxt.at[pl.ds(off, 128)]],
                    rb.at[pl.ds(j * 128, 128)], sem))
            for d in descs:
                d.wait()
            pltpu.sync_copy(ra, g1_hbm.at[pl.ds(base + ci * CH, CH)])
            pltpu.sync_copy(rb, g2_hbm.at[pl.ds(base + ci * CH, CH)])
            return carry

        lax.fori_loop(0, NCH, chunk, 0, unroll=False)

    return k(t_tab, src, tgt)


_BAND = 512
_PB = _BAND * N
_TSH = _PB // 16
_DUMP = 4096


def _sc_scatter(src, tgt, w):
    mesh = plsc.VectorSubcoreMesh(core_axis_name="c", subcore_axis_name="s")

    @functools.partial(
        pl.kernel,
        out_type=[jax.ShapeDtypeStruct((N * N,), jnp.float32),
                  jax.ShapeDtypeStruct((N,), jnp.float32)],
        mesh=mesh,
        scratch_types=[
            pltpu.VMEM_SHARED((_PB + _DUMP,), jnp.float32),
            pltpu.VMEM_SHARED((N,), jnp.float32),
            pltpu.VMEM((8192,), jnp.float32),
            pltpu.VMEM((12288,), jnp.float32),
            pltpu.VMEM((1024,), jnp.int32),
            pltpu.VMEM((1024,), jnp.int32),
            pltpu.VMEM((1024,), jnp.float32),
            pltpu.VMEM((8, 128), jnp.int32),
            pltpu.VMEM((8, 128), jnp.float32),
            pltpu.VMEM((8, 128), jnp.int32),
            pltpu.VMEM((8, 128), jnp.float32),
            pltpu.SemaphoreType.DMA,
        ],
    )
    def k(src_hbm, tgt_hbm, w_hbm, adj_hbm, rs_hbm,
          band_sp, rs_sp, zbuf, bb, cs, ct, cw, oidx, oval, ridx, rval, sem):
        c = lax.axis_index("c")
        s = lax.axis_index("s")
        lane = lax.iota(jnp.int32, 16)

        def zb(i, carry):
            zbuf[pl.ds(i * 16, 16)] = jnp.zeros((16,), jnp.float32)
            return carry
        lax.fori_loop(0, 8192 // 16, zb, 0, unroll=False)

        for p in range(3):
            lo = (c * 3 + p) * _BAND

            def zcp(t, carry):
                pltpu.sync_copy(zbuf,
                                band_sp.at[pl.ds(s * _TSH + t * 8192, 8192)])
                return carry
            lax.fori_loop(0, _TSH // 8192, zcp, 0, unroll=False)

            @pl.when(s == 0)
            def _():
                pltpu.sync_copy(zbuf.at[pl.ds(0, _DUMP)], band_sp.at[pl.ds(_PB, _DUMP)])

            if p == 0:
                @pl.when((s == 1) & (c == 0))
                def _():
                    pltpu.sync_copy(zbuf.at[pl.ds(0, N)], rs_sp)

            plsc.subcore_barrier()

            def chunk(ci, carry):
                eb = (s * 6 + ci) * 1024
                pltpu.sync_copy(src_hbm.at[pl.ds(eb, 1024)], cs)
                pltpu.sync_copy(tgt_hbm.at[pl.ds(eb, 1024)], ct)
                pltpu.sync_copy(w_hbm.at[pl.ds(eb, 1024)], cw)
                dump_base = _PB + s * 256
                for g in range(64):
                    r, col = g // 8, (g % 8) * 16
                    s16 = cs[pl.ds(g * 16, 16)]
                    t16 = ct[pl.ds(g * 16, 16)]
                    w16 = cw[pl.ds(g * 16, 16)]
                    m = (s16 >= lo) & (s16 < lo + _BAND)
                    off = (s16 - lo) * N + t16
                    dmp = dump_base + (g % 16) * 16 + lane
                    oidx[r, pl.ds(col, 16)] = jnp.where(m, off, dmp)
                    oval[r, pl.ds(col, 16)] = jnp.where(m, w16, 0.0)
                    if p == 0:
                        ridx[r, pl.ds(col, 16)] = s16
                        rval[r, pl.ds(col, 16)] = w16
                descs = [pltpu.async_copy(oval.at[r], band_sp.at[oidx.at[r]],
                                          sem, add=True) for r in range(8)]
                if p == 0:
                    @pl.when(c == 0)
                    def _():
                        d2 = [pltpu.async_copy(rval.at[r], rs_sp.at[ridx.at[r]],
                                               sem, add=True) for r in range(8)]
                        for d in d2:
                            d.wait()
                for d in descs:
                    d.wait()
                return carry

            lax.fori_loop(0, 6, chunk, 0, unroll=False)
            plsc.subcore_barrier()

            adj_base = (c * 3 + p) * _PB + s * _TSH

            def ocp(t, carry):
                pltpu.sync_copy(band_sp.at[pl.ds(s * _TSH + t * 12288, 12288)], bb)
                pltpu.sync_copy(bb, adj_hbm.at[pl.ds(adj_base + t * 12288, 12288)])
                return carry
            lax.fori_loop(0, _TSH // 12288, ocp, 0, unroll=False)
            if p == 0:
                @pl.when(c == 0)
                def _():
                    pltpu.sync_copy(rs_sp.at[pl.ds(s * 192, 192)],
                                    bb.at[pl.ds(0, 192)])
                    pltpu.sync_copy(bb.at[pl.ds(0, 192)],
                                    rs_hbm.at[pl.ds(s * 192, 192)])
            plsc.subcore_barrier()

    return k(src, tgt, w)


def kernel(feature_tuple, dia_lens, win_p, win_f, edge_index, ewg_proj_W,
           mlp_W1, mlp_b1, mlp_W2, mlp_b2, l0_W1, l0_W2, l1_W1, l1_W2,
           *, interpret=False):
    feat = feature_tuple.reshape(-1, feature_tuple.shape[-1])
    src, tgt = edge_index[0], edge_index[1]

    wpT = ewg_proj_W.T
    w1aT = mlp_W1[:, :SH].T
    w1bT = mlp_W1[:, SH:].T
    b1 = mlp_b1.reshape(1, SH)
    t_tab = _edge_prep(feat, wpT, w1aT, w1bT, b1, interpret=interpret)

    if interpret:
        g1, g2 = t_tab[src], t_tab[tgt]
    else:
        g1, g2 = _sc_gather(t_tab, src, tgt)

    w = _edge_mlp(g1, g2, mlp_W2.reshape(1, SH), mlp_b2, interpret=interpret)

    if interpret:
        adjv = jnp.zeros((N * N,), jnp.float32).at[src * N + tgt].add(w)
        rs = jnp.zeros((N,), jnp.float32).at[src].add(w)
    else:
        adjv, rs = _sc_scatter(src, tgt, w)
    adj = adjv.reshape(N, N)
    rs2d = rs.reshape(N, 1)

    x = feat
    for (W1, W2) in [(l0_W1, l0_W2), (l1_W1, l1_W2)]:
        q = _q_step(x, W1.T, rs2d, interpret=interpret)
        x = _layer_step(adj, q, x, rs2d, W2[:, :D].T, W2[:, D:].T,
                        interpret=interpret)

    f0, f1, f2 = jnp.split(x, 3, axis=0)
    return (f0, f1, f2, edge_index)


# speedup vs baseline: 3.2287x; 1.0507x over previous
import functools

import jax
import jax.numpy as jnp
from jax import lax
from jax.experimental import pallas as pl
from jax.experimental.pallas import tpu as pltpu
from jax.experimental.pallas import tpu_sc as plsc

N = 3072
D = 256
SH = 64
E = 98304


def _leaky(x):
    return jnp.where(x >= 0, x, 0.01 * x)


def _edge_prep_body(feat_ref, wpT_ref, w1aT_ref, w1bT_ref, b1_ref, t_ref):
    p = jnp.dot(feat_ref[...], wpT_ref[...], preferred_element_type=jnp.float32)
    nrm = jnp.sqrt(jnp.sum(p * p, axis=1, keepdims=True))
    pn = p / jnp.maximum(nrm, 1e-12)
    a = jnp.dot(pn, w1aT_ref[...], preferred_element_type=jnp.float32) + b1_ref[...]
    b = jnp.dot(pn, w1bT_ref[...], preferred_element_type=jnp.float32)
    t_ref[...] = jnp.concatenate([a, b], axis=-1)


def _edge_prep(feat, wpT, w1aT, w1bT, b1, *, interpret=False):
    BM = 512
    grid = (N // BM,)
    return pl.pallas_call(
        _edge_prep_body,
        grid=grid,
        in_specs=[
            pl.BlockSpec((BM, D), lambda i: (i, 0)),
            pl.BlockSpec((D, SH), lambda i: (0, 0)),
            pl.BlockSpec((SH, SH), lambda i: (0, 0)),
            pl.BlockSpec((SH, SH), lambda i: (0, 0)),
            pl.BlockSpec((1, SH), lambda i: (0, 0)),
        ],
        out_specs=pl.BlockSpec((BM, 2 * SH), lambda i: (i, 0)),
        out_shape=jax.ShapeDtypeStruct((N, 2 * SH), jnp.float32),
        interpret=interpret,
    )(feat, wpT, w1aT, w1bT, b1)


def _edge_mlp_body(h_ref, w2_ref, b2_ref, w_ref):
    l = _leaky(h_ref[...])
    s = jnp.sum(l * w2_ref[...], axis=1) + b2_ref[0]
    w_ref[...] = jax.nn.sigmoid(s)


def _edge_mlp(h, w2row, b2, *, interpret=False):
    BE = 12288
    grid = (E // BE,)
    return pl.pallas_call(
        _edge_mlp_body,
        grid=grid,
        in_specs=[
            pl.BlockSpec((BE, SH), lambda i: (i, 0)),
            pl.BlockSpec((1, SH), lambda i: (0, 0)),
            pl.BlockSpec(memory_space=pltpu.SMEM),
        ],
        out_specs=pl.BlockSpec((BE,), lambda i: (i,)),
        out_shape=jax.ShapeDtypeStruct((E,), jnp.float32),
        interpret=interpret,
    )(h, w2row, b2)


def _q_body(x_ref, w1T_ref, rs_ref, q_ref):
    i = pl.program_id(0)
    rs = rs_ref[pl.ds(i * 512, 512), :]
    d = jnp.where(rs > 0, lax.rsqrt(jnp.where(rs > 0, rs, 1.0)), 0.0)
    q_ref[...] = jnp.dot(x_ref[...], w1T_ref[...], preferred_element_type=jnp.float32) * d


def _q_step(x, w1T, rs2d, *, interpret=False):
    BM = 512
    grid = (N // BM,)
    return pl.pallas_call(
        _q_body,
        grid=grid,
        in_specs=[
            pl.BlockSpec((BM, D), lambda i: (i, 0)),
            pl.BlockSpec((D, D), lambda i: (0, 0)),
            pl.BlockSpec((N, 1), lambda i: (0, 0)),
        ],
        out_specs=pl.BlockSpec((BM, D), lambda i: (i, 0)),
        out_shape=jax.ShapeDtypeStruct((N, D), jnp.float32),
        interpret=interpret,
    )(x, w1T, rs2d)


def _layer_body(adj_ref, q_ref, x_ref, rs_ref, w2aT_ref, w2bT_ref, out_ref, acc_ref):
    m = pl.program_id(0)
    k = pl.program_id(1)
    nk = pl.num_programs(1)
    part = jnp.dot(adj_ref[...], q_ref[...], preferred_element_type=jnp.float32)

    @pl.when(k == 0)
    def _():
        acc_ref[...] = part

    @pl.when(k != 0)
    def _():
        acc_ref[...] += part

    @pl.when(k == nk - 1)
    def _():
        rs = rs_ref[pl.ds(m * 512, 512), :]
        d = jnp.where(rs > 0, lax.rsqrt(jnp.where(rs > 0, rs, 1.0)), 0.0)
        nb = acc_ref[...] * d
        x = x_ref[...]
        u = x + nb
        v = x * nb
        out_ref[...] = _leaky(
            jnp.dot(u, w2aT_ref[...], preferred_element_type=jnp.float32)
            + jnp.dot(v, w2bT_ref[...], preferred_element_type=jnp.float32))


def _layer_step(adj, q, x, rs2d, w2aT, w2bT, *, interpret=False):
    BM, BK = 512, 512
    grid = (N // BM, N // BK)
    return pl.pallas_call(
        _layer_body,
        grid=grid,
        in_specs=[
            pl.BlockSpec((BM, BK), lambda m, k: (m, k)),
            pl.BlockSpec((BK, D), lambda m, k: (k, 0)),
            pl.BlockSpec((BM, D), lambda m, k: (m, 0)),
            pl.BlockSpec((N, 1), lambda m, k: (0, 0)),
            pl.BlockSpec((D, D), lambda m, k: (0, 0)),
            pl.BlockSpec((D, D), lambda m, k: (0, 0)),
        ],
        out_specs=pl.BlockSpec((BM, D), lambda m, k: (m, 0)),
        out_shape=jax.ShapeDtypeStruct((N, D), jnp.float32),
        scratch_shapes=[pltpu.VMEM((BM, D), jnp.float32)],
        interpret=interpret,
    )(adj, q, x, rs2d, w2aT, w2bT)


def _sc_gather(t_tab, src, tgt):
    EW = E // 32
    CH = 128
    NCH = EW // CH

    mesh = plsc.VectorSubcoreMesh(core_axis_name="c", subcore_axis_name="s")

    @functools.partial(
        pl.kernel,
        out_type=jax.ShapeDtypeStruct((E, SH), jnp.float32),
        mesh=mesh,
        scratch_types=[
            pltpu.VMEM((EW,), jnp.int32),
            pltpu.VMEM((EW,), jnp.int32),
            pltpu.VMEM((CH, 2 * SH), jnp.float32),
            pltpu.VMEM((CH, 2 * SH), jnp.float32),
            pltpu.VMEM((CH, 2 * SH), jnp.float32),
            pltpu.VMEM((CH, 2 * SH), jnp.float32),
            pltpu.VMEM((CH, SH), jnp.float32),
            pltpu.VMEM((CH, SH), jnp.float32),
            pltpu.SemaphoreType.DMA,
            pltpu.SemaphoreType.DMA,
        ],
    )
    def k(t_hbm, src_hbm, tgt_hbm, h_hbm,
          idxs, idxt, ra0, ra1, rb0, rb1, h0, h1, semg, semw):
        ra = (ra0, ra1)
        rb = (rb0, rb1)
        hb = (h0, h1)
        wid = lax.axis_index("s") * 2 + lax.axis_index("c")
        base = wid * EW
        pltpu.sync_copy(src_hbm.at[pl.ds(base, EW)], idxs)
        pltpu.sync_copy(tgt_hbm.at[pl.ds(base, EW)], idxt)

        def fire(ci, b):
            pltpu.async_copy(t_hbm.at[idxs.at[pl.ds(ci * CH, CH)]], ra[b], semg)
            pltpu.async_copy(t_hbm.at[idxt.at[pl.ds(ci * CH, CH)]], rb[b], semg)

        def wait_gather(ci, b):
            pltpu.make_async_copy(
                t_hbm.at[idxs.at[pl.ds(ci * CH, CH)]], ra[b], semg).wait()
            pltpu.make_async_copy(
                t_hbm.at[idxt.at[pl.ds(ci * CH, CH)]], rb[b], semg).wait()

        def wdesc(ci, b):
            return pltpu.make_async_copy(
                hb[b], h_hbm.at[pl.ds(base + ci * CH, CH)], semw)

        fire(0, 0)
        for ci in range(NCH):
            b = ci & 1
            if ci + 1 < NCH:
                fire(ci + 1, 1 - b)
            wait_gather(ci, b)
            if ci >= 2:
                wdesc(ci - 2, b).wait()

            def add_edge(e, carry):
                for g in range(SH // 16):
                    hb[b][e, pl.ds(g * 16, 16)] = (
                        ra[b][e, pl.ds(g * 16, 16)]
                        + rb[b][e, pl.ds(SH + g * 16, 16)])
                return carry
            lax.fori_loop(0, CH, add_edge, 0, unroll=4)

            pltpu.async_copy(hb[b], h_hbm.at[pl.ds(base + ci * CH, CH)], semw)
        wdesc(NCH - 2, 0).wait()
        wdesc(NCH - 1, 1).wait()

    return k(t_tab, src, tgt)


_BAND = 512
_PB = _BAND * N
_TSH = _PB // 16
_DUMP = 4096


def _sc_scatter(src, tgt, w):
    mesh = plsc.VectorSubcoreMesh(core_axis_name="c", subcore_axis_name="s")

    @functools.partial(
        pl.kernel,
        out_type=[jax.ShapeDtypeStruct((N * N,), jnp.float32),
                  jax.ShapeDtypeStruct((N,), jnp.float32)],
        mesh=mesh,
        scratch_types=[
            pltpu.VMEM_SHARED((_PB + _DUMP,), jnp.float32),
            pltpu.VMEM_SHARED((N,), jnp.float32),
            pltpu.VMEM((8192,), jnp.float32),
            pltpu.VMEM((12288,), jnp.float32),
            pltpu.VMEM((1024,), jnp.int32),
            pltpu.VMEM((1024,), jnp.int32),
            pltpu.VMEM((1024,), jnp.float32),
            pltpu.VMEM((8, 128), jnp.int32),
            pltpu.VMEM((8, 128), jnp.float32),
            pltpu.VMEM((8, 128), jnp.int32),
            pltpu.VMEM((8, 128), jnp.float32),
            pltpu.SemaphoreType.DMA,
        ],
    )
    def k(src_hbm, tgt_hbm, w_hbm, adj_hbm, rs_hbm,
          band_sp, rs_sp, zbuf, bb, cs, ct, cw, oidx, oval, ridx, rval, sem):
        c = lax.axis_index("c")
        s = lax.axis_index("s")
        lane = lax.iota(jnp.int32, 16)

        def zb(i, carry):
            zbuf[pl.ds(i * 16, 16)] = jnp.zeros((16,), jnp.float32)
            return carry
        lax.fori_loop(0, 8192 // 16, zb, 0, unroll=False)

        for p in range(3):
            lo = (c * 3 + p) * _BAND

            def zcp(t, carry):
                pltpu.sync_copy(zbuf,
                                band_sp.at[pl.ds(s * _TSH + t * 8192, 8192)])
                return carry
            lax.fori_loop(0, _TSH // 8192, zcp, 0, unroll=False)

            @pl.when(s == 0)
            def _():
                pltpu.sync_copy(zbuf.at[pl.ds(0, _DUMP)], band_sp.at[pl.ds(_PB, _DUMP)])

            if p == 0:
                @pl.when((s == 1) & (c == 0))
                def _():
                    pltpu.sync_copy(zbuf.at[pl.ds(0, N)], rs_sp)

            plsc.subcore_barrier()

            def chunk(ci, carry):
                eb = (s * 6 + ci) * 1024
                pltpu.sync_copy(src_hbm.at[pl.ds(eb, 1024)], cs)
                pltpu.sync_copy(tgt_hbm.at[pl.ds(eb, 1024)], ct)
                pltpu.sync_copy(w_hbm.at[pl.ds(eb, 1024)], cw)
                dump_base = _PB + s * 256
                for g in range(64):
                    r, col = g // 8, (g % 8) * 16
                    s16 = cs[pl.ds(g * 16, 16)]
                    t16 = ct[pl.ds(g * 16, 16)]
                    w16 = cw[pl.ds(g * 16, 16)]
                    m = (s16 >= lo) & (s16 < lo + _BAND)
                    off = (s16 - lo) * N + t16
                    dmp = dump_base + (g % 16) * 16 + lane
                    oidx[r, pl.ds(col, 16)] = jnp.where(m, off, dmp)
                    oval[r, pl.ds(col, 16)] = jnp.where(m, w16, 0.0)
                    if p == 0:
                        ridx[r, pl.ds(col, 16)] = s16
                        rval[r, pl.ds(col, 16)] = w16
                descs = [pltpu.async_copy(oval.at[r], band_sp.at[oidx.at[r]],
                                          sem, add=True) for r in range(8)]
                if p == 0:
                    @pl.when(c == 0)
                    def _():
                        d2 = [pltpu.async_copy(rval.at[r], rs_sp.at[ridx.at[r]],
                                               sem, add=True) for r in range(8)]
                        for d in d2:
                            d.wait()
                for d in descs:
                    d.wait()
                return carry

            lax.fori_loop(0, 6, chunk, 0, unroll=False)
            plsc.subcore_barrier()

            adj_base = (c * 3 + p) * _PB + s * _TSH

            def ocp(t, carry):
                pltpu.sync_copy(band_sp.at[pl.ds(s * _TSH + t * 12288, 12288)], bb)
                pltpu.sync_copy(bb, adj_hbm.at[pl.ds(adj_base + t * 12288, 12288)])
                return carry
            lax.fori_loop(0, _TSH // 12288, ocp, 0, unroll=False)
            if p == 0:
                @pl.when(c == 0)
                def _():
                    pltpu.sync_copy(rs_sp.at[pl.ds(s * 192, 192)],
                                    bb.at[pl.ds(0, 192)])
                    pltpu.sync_copy(bb.at[pl.ds(0, 192)],
                                    rs_hbm.at[pl.ds(s * 192, 192)])
            plsc.subcore_barrier()

    return k(src, tgt, w)


def kernel(feature_tuple, dia_lens, win_p, win_f, edge_index, ewg_proj_W,
           mlp_W1, mlp_b1, mlp_W2, mlp_b2, l0_W1, l0_W2, l1_W1, l1_W2,
           *, interpret=False):
    feat = feature_tuple.reshape(-1, feature_tuple.shape[-1])
    src, tgt = edge_index[0], edge_index[1]

    wpT = ewg_proj_W.T
    w1aT = mlp_W1[:, :SH].T
    w1bT = mlp_W1[:, SH:].T
    b1 = mlp_b1.reshape(1, SH)
    t_tab = _edge_prep(feat, wpT, w1aT, w1bT, b1, interpret=interpret)

    if interpret:
        h = t_tab[src][:, :SH] + t_tab[tgt][:, SH:]
    else:
        h = _sc_gather(t_tab, src, tgt)

    w = _edge_mlp(h, mlp_W2.reshape(1, SH), mlp_b2, interpret=interpret)

    if interpret:
        adjv = jnp.zeros((N * N,), jnp.float32).at[src * N + tgt].add(w)
        rs = jnp.zeros((N,), jnp.float32).at[src].add(w)
    else:
        adjv, rs = _sc_scatter(src, tgt, w)
    adj = adjv.reshape(N, N)
    rs2d = rs.reshape(N, 1)

    x = feat
    for (W1, W2) in [(l0_W1, l0_W2), (l1_W1, l1_W2)]:
        q = _q_step(x, W1.T, rs2d, interpret=interpret)
        x = _layer_step(adj, q, x, rs2d, W2[:, :D].T, W2[:, D:].T,
                        interpret=interpret)

    f0, f1, f2 = jnp.split(x, 3, axis=0)
    return (f0, f1, f2, edge_index)


# speedup vs baseline: 3.4573x; 1.0708x over previous
import functools

import jax
import jax.numpy as jnp
from jax import lax
from jax.experimental import pallas as pl
from jax.experimental.pallas import tpu as pltpu
from jax.experimental.pallas import tpu_sc as plsc

N = 3072
D = 256
SH = 64
E = 98304


def _leaky(x):
    return jnp.where(x >= 0, x, 0.01 * x)


def _edge_prep_body(feat_ref, wpT_ref, w1aT_ref, w1bT_ref, b1_ref, t_ref):
    p = jnp.dot(feat_ref[...], wpT_ref[...], preferred_element_type=jnp.float32)
    nrm = jnp.sqrt(jnp.sum(p * p, axis=1, keepdims=True))
    pn = p / jnp.maximum(nrm, 1e-12)
    a = jnp.dot(pn, w1aT_ref[...], preferred_element_type=jnp.float32) + b1_ref[...]
    b = jnp.dot(pn, w1bT_ref[...], preferred_element_type=jnp.float32)
    t_ref[...] = jnp.concatenate([a, b], axis=-1)


def _edge_prep(feat, wpT, w1aT, w1bT, b1, *, interpret=False):
    BM = 512
    grid = (N // BM,)
    return pl.pallas_call(
        _edge_prep_body,
        grid=grid,
        in_specs=[
            pl.BlockSpec((BM, D), lambda i: (i, 0)),
            pl.BlockSpec((D, SH), lambda i: (0, 0)),
            pl.BlockSpec((SH, SH), lambda i: (0, 0)),
            pl.BlockSpec((SH, SH), lambda i: (0, 0)),
            pl.BlockSpec((1, SH), lambda i: (0, 0)),
        ],
        out_specs=pl.BlockSpec((BM, 2 * SH), lambda i: (i, 0)),
        out_shape=jax.ShapeDtypeStruct((N, 2 * SH), jnp.float32),
        interpret=interpret,
    )(feat, wpT, w1aT, w1bT, b1)


def _edge_mlp_body(h_ref, w2_ref, b2_ref, w_ref):
    l = _leaky(h_ref[...])
    s = jnp.dot(l, w2_ref[...], preferred_element_type=jnp.float32)
    w_ref[...] = jax.nn.sigmoid(s[:, 0] + b2_ref[0])


def _edge_mlp(h, w2col, b2, *, interpret=False):
    BE = 12288
    grid = (E // BE,)
    return pl.pallas_call(
        _edge_mlp_body,
        grid=grid,
        in_specs=[
            pl.BlockSpec((BE, SH), lambda i: (i, 0)),
            pl.BlockSpec((SH, 1), lambda i: (0, 0)),
            pl.BlockSpec(memory_space=pltpu.SMEM),
        ],
        out_specs=pl.BlockSpec((BE,), lambda i: (i,)),
        out_shape=jax.ShapeDtypeStruct((E,), jnp.float32),
        interpret=interpret,
    )(h, w2col, b2)


def _q_body(x_ref, w1T_ref, rs_ref, q_ref):
    i = pl.program_id(0)
    rs = rs_ref[pl.ds(i * 512, 512), :]
    d = jnp.where(rs > 0, lax.rsqrt(jnp.where(rs > 0, rs, 1.0)), 0.0)
    q_ref[...] = jnp.dot(x_ref[...], w1T_ref[...], preferred_element_type=jnp.float32) * d


def _q_step(x, w1T, rs2d, *, interpret=False):
    BM = 512
    grid = (N // BM,)
    return pl.pallas_call(
        _q_body,
        grid=grid,
        in_specs=[
            pl.BlockSpec((BM, D), lambda i: (i, 0)),
            pl.BlockSpec((D, D), lambda i: (0, 0)),
            pl.BlockSpec((N, 1), lambda i: (0, 0)),
        ],
        out_specs=pl.BlockSpec((BM, D), lambda i: (i, 0)),
        out_shape=jax.ShapeDtypeStruct((N, D), jnp.float32),
        interpret=interpret,
    )(x, w1T, rs2d)


def _layer_body(adj_ref, q_ref, x_ref, rs_ref, w2aT_ref, w2bT_ref, out_ref, acc_ref):
    m = pl.program_id(0)
    k = pl.program_id(1)
    nk = pl.num_programs(1)
    part = jnp.dot(adj_ref[...], q_ref[...], preferred_element_type=jnp.float32)

    @pl.when(k == 0)
    def _():
        acc_ref[...] = part

    @pl.when(k != 0)
    def _():
        acc_ref[...] += part

    @pl.when(k == nk - 1)
    def _():
        rs = rs_ref[pl.ds(m * 512, 512), :]
        d = jnp.where(rs > 0, lax.rsqrt(jnp.where(rs > 0, rs, 1.0)), 0.0)
        nb = acc_ref[...] * d
        x = x_ref[...]
        u = x + nb
        v = x * nb
        out_ref[...] = _leaky(
            jnp.dot(u, w2aT_ref[...], preferred_element_type=jnp.float32)
            + jnp.dot(v, w2bT_ref[...], preferred_element_type=jnp.float32))


def _layer_step(adj, q, x, rs2d, w2aT, w2bT, *, interpret=False):
    BM, BK = 512, 512
    grid = (N // BM, N // BK)
    return pl.pallas_call(
        _layer_body,
        grid=grid,
        in_specs=[
            pl.BlockSpec((BM, BK), lambda m, k: (m, k)),
            pl.BlockSpec((BK, D), lambda m, k: (k, 0)),
            pl.BlockSpec((BM, D), lambda m, k: (m, 0)),
            pl.BlockSpec((N, 1), lambda m, k: (0, 0)),
            pl.BlockSpec((D, D), lambda m, k: (0, 0)),
            pl.BlockSpec((D, D), lambda m, k: (0, 0)),
        ],
        out_specs=pl.BlockSpec((BM, D), lambda m, k: (m, 0)),
        out_shape=jax.ShapeDtypeStruct((N, D), jnp.float32),
        scratch_shapes=[pltpu.VMEM((BM, D), jnp.float32)],
        interpret=interpret,
    )(adj, q, x, rs2d, w2aT, w2bT)


def _sc_gather(t_tab, src, tgt):
    EW = E // 32
    CH = 128
    NCH = EW // CH

    mesh = plsc.VectorSubcoreMesh(core_axis_name="c", subcore_axis_name="s")

    @functools.partial(
        pl.kernel,
        out_type=jax.ShapeDtypeStruct((E, SH), jnp.float32),
        mesh=mesh,
        scratch_types=[
            pltpu.VMEM((EW,), jnp.int32),
            pltpu.VMEM((EW,), jnp.int32),
            pltpu.VMEM((CH, 2 * SH), jnp.float32),
            pltpu.VMEM((CH, 2 * SH), jnp.float32),
            pltpu.VMEM((CH, 2 * SH), jnp.float32),
            pltpu.VMEM((CH, 2 * SH), jnp.float32),
            pltpu.VMEM((CH, SH), jnp.float32),
            pltpu.VMEM((CH, SH), jnp.float32),
            pltpu.SemaphoreType.DMA,
            pltpu.SemaphoreType.DMA,
        ],
    )
    def k(t_hbm, src_hbm, tgt_hbm, h_hbm,
          idxs, idxt, ra0, ra1, rb0, rb1, h0, h1, semg, semw):
        ra = (ra0, ra1)
        rb = (rb0, rb1)
        hb = (h0, h1)
        wid = lax.axis_index("s") * 2 + lax.axis_index("c")
        base = wid * EW
        pltpu.sync_copy(src_hbm.at[pl.ds(base, EW)], idxs)
        pltpu.sync_copy(tgt_hbm.at[pl.ds(base, EW)], idxt)

        def fire(ci, b):
            pltpu.async_copy(t_hbm.at[idxs.at[pl.ds(ci * CH, CH)]], ra[b], semg)
            pltpu.async_copy(t_hbm.at[idxt.at[pl.ds(ci * CH, CH)]], rb[b], semg)

        def wait_gather(ci, b):
            pltpu.make_async_copy(
                t_hbm.at[idxs.at[pl.ds(ci * CH, CH)]], ra[b], semg).wait()
            pltpu.make_async_copy(
                t_hbm.at[idxt.at[pl.ds(ci * CH, CH)]], rb[b], semg).wait()

        def wdesc(ci, b):
            return pltpu.make_async_copy(
                hb[b], h_hbm.at[pl.ds(base + ci * CH, CH)], semw)

        fire(0, 0)
        for ci in range(NCH):
            b = ci & 1
            if ci + 1 < NCH:
                fire(ci + 1, 1 - b)
            wait_gather(ci, b)
            if ci >= 2:
                wdesc(ci - 2, b).wait()

            def add_edge(e, carry):
                for g in range(SH // 16):
                    hb[b][e, pl.ds(g * 16, 16)] = (
                        ra[b][e, pl.ds(g * 16, 16)]
                        + rb[b][e, pl.ds(SH + g * 16, 16)])
                return carry
            lax.fori_loop(0, CH, add_edge, 0, unroll=4)

            pltpu.async_copy(hb[b], h_hbm.at[pl.ds(base + ci * CH, CH)], semw)
        wdesc(NCH - 2, 0).wait()
        wdesc(NCH - 1, 1).wait()

    return k(t_tab, src, tgt)


_BAND = 384
_NPASS = 4
_PB = _BAND * N
_TSH = _PB // 16
_DUMP = 4096


def _sc_scatter(src, tgt, w):
    mesh = plsc.VectorSubcoreMesh(core_axis_name="c", subcore_axis_name="s")

    @functools.partial(
        pl.kernel,
        out_type=[jax.ShapeDtypeStruct((N, N), jnp.float32),
                  jax.ShapeDtypeStruct((N,), jnp.float32)],
        mesh=mesh,
        scratch_types=[
            pltpu.VMEM_SHARED((_PB + _DUMP,), jnp.float32),
            pltpu.VMEM_SHARED((N,), jnp.float32),
            pltpu.VMEM((8192,), jnp.float32),
            pltpu.VMEM((8, N), jnp.float32),
            pltpu.VMEM((1024,), jnp.int32),
            pltpu.VMEM((1024,), jnp.int32),
            pltpu.VMEM((1024,), jnp.float32),
            pltpu.VMEM((8, 128), jnp.int32),
            pltpu.VMEM((8, 128), jnp.float32),
            pltpu.VMEM((8, 128), jnp.int32),
            pltpu.VMEM((8, 128), jnp.float32),
            pltpu.SemaphoreType.DMA,
        ],
    )
    def k(src_hbm, tgt_hbm, w_hbm, adj_hbm, rs_hbm,
          band_sp, rs_sp, zbuf, bb, cs, ct, cw, oidx, oval, ridx, rval, sem):
        c = lax.axis_index("c")
        s = lax.axis_index("s")
        lane = lax.iota(jnp.int32, 16)

        def zb(i, carry):
            zbuf[pl.ds(i * 16, 16)] = jnp.zeros((16,), jnp.float32)
            return carry
        lax.fori_loop(0, 8192 // 16, zb, 0, unroll=False)

        def one_pass(p, pcarry):
            band = c * _NPASS + p
            lo = band * _BAND

            def zcp(t, carry):
                pltpu.sync_copy(zbuf,
                                band_sp.at[pl.ds(s * _TSH + t * 8192, 8192)])
                return carry
            lax.fori_loop(0, _TSH // 8192, zcp, 0, unroll=False)

            @pl.when(s == 0)
            def _():
                pltpu.sync_copy(zbuf.at[pl.ds(0, _DUMP)], band_sp.at[pl.ds(_PB, _DUMP)])

            @pl.when((p == 0) & (s == 1) & (c == 0))
            def _():
                pltpu.sync_copy(zbuf.at[pl.ds(0, 3072)], rs_sp)

            plsc.subcore_barrier()

            def chunk(ci, carry):
                eb = (s * 6 + ci) * 1024
                pltpu.sync_copy(src_hbm.at[pl.ds(eb, 1024)], cs)
                pltpu.sync_copy(tgt_hbm.at[pl.ds(eb, 1024)], ct)
                pltpu.sync_copy(w_hbm.at[pl.ds(eb, 1024)], cw)
                dump_base = _PB + s * 256
                for g in range(64):
                    r, col = g // 8, (g % 8) * 16
                    s16 = cs[pl.ds(g * 16, 16)]
                    t16 = ct[pl.ds(g * 16, 16)]
                    w16 = cw[pl.ds(g * 16, 16)]
                    m = (s16 >= lo) & (s16 < lo + _BAND)
                    off = (s16 - lo) * N + t16
                    dmp = dump_base + (g % 16) * 16 + lane
                    oidx[r, pl.ds(col, 16)] = jnp.where(m, off, dmp)
                    oval[r, pl.ds(col, 16)] = jnp.where(m, w16, 0.0)
                    ridx[r, pl.ds(col, 16)] = s16
                    rval[r, pl.ds(col, 16)] = w16
                descs = [pltpu.async_copy(oval.at[r], band_sp.at[oidx.at[r]],
                                          sem, add=True) for r in range(8)]

                @pl.when((p == 0) & (c == 0))
                def _():
                    d2 = [pltpu.async_copy(rval.at[r], rs_sp.at[ridx.at[r]],
                                           sem, add=True) for r in range(8)]
                    for d in d2:
                        d.wait()
                for d in descs:
                    d.wait()
                return carry

            lax.fori_loop(0, 6, chunk, 0, unroll=False)
            plsc.subcore_barrier()

            row0 = lo + s * (_TSH // N)

            def ocp(t, carry):
                base_sp = s * _TSH + t * 8 * N
                ds8 = [pltpu.async_copy(band_sp.at[pl.ds(base_sp + r * N, N)],
                                        bb.at[r], sem) for r in range(8)]
                for d in ds8:
                    d.wait()
                pltpu.sync_copy(bb, adj_hbm.at[pl.ds(row0 + t * 8, 8)])
                return carry
            lax.fori_loop(0, _TSH // (8 * N), ocp, 0, unroll=False)

            @pl.when((p == 0) & (c == 0))
            def _():
                pltpu.sync_copy(rs_sp.at[pl.ds(s * 192, 192)],
                                cw.at[pl.ds(0, 192)])
                pltpu.sync_copy(cw.at[pl.ds(0, 192)],
                                rs_hbm.at[pl.ds(s * 192, 192)])
            plsc.subcore_barrier()
            return pcarry

        lax.fori_loop(0, _NPASS, one_pass, 0, unroll=False)

    return k(src, tgt, w)


def kernel(feature_tuple, dia_lens, win_p, win_f, edge_index, ewg_proj_W,
           mlp_W1, mlp_b1, mlp_W2, mlp_b2, l0_W1, l0_W2, l1_W1, l1_W2,
           *, interpret=False):
    feat = feature_tuple.reshape(-1, feature_tuple.shape[-1])
    src, tgt = edge_index[0], edge_index[1]

    wpT = ewg_proj_W.T
    w1aT = mlp_W1[:, :SH].T
    w1bT = mlp_W1[:, SH:].T
    b1 = mlp_b1.reshape(1, SH)
    t_tab = _edge_prep(feat, wpT, w1aT, w1bT, b1, interpret=interpret)

    if interpret:
        h = t_tab[src][:, :SH] + t_tab[tgt][:, SH:]
    else:
        h = _sc_gather(t_tab, src, tgt)

    w = _edge_mlp(h, mlp_W2.reshape(SH, 1), mlp_b2, interpret=interpret)

    if interpret:
        adj = jnp.zeros((N, N), jnp.float32).at[src, tgt].add(w)
        rs = jnp.zeros((N,), jnp.float32).at[src].add(w)
    else:
        adj, rs = _sc_scatter(src, tgt, w)
    rs2d = rs.reshape(N, 1)

    x = feat
    for (W1, W2) in [(l0_W1, l0_W2), (l1_W1, l1_W2)]:
        q = _q_step(x, W1.T, rs2d, interpret=interpret)
        x = _layer_step(adj, q, x, rs2d, W2[:, :D].T, W2[:, D:].T,
                        interpret=interpret)

    f0, f1, f2 = jnp.split(x, 3, axis=0)
    return (f0, f1, f2, edge_index)


# speedup vs baseline: 3.7318x; 1.0794x over previous
import functools

import jax
import jax.numpy as jnp
import numpy as np
from jax import lax
from jax.experimental import pallas as pl
from jax.experimental.pallas import tpu as pltpu
from jax.experimental.pallas import tpu_sc as plsc

N = 3072
D = 256
SH = 64
E = 98304


def _leaky(x):
    return jnp.where(x >= 0, x, 0.01 * x)


def _edge_prep_body(feat_ref, wpT_ref, w1aT_ref, w1bT_ref, b1_ref, t_ref):
    p = jnp.dot(feat_ref[...], wpT_ref[...], preferred_element_type=jnp.float32)
    nrm = jnp.sqrt(jnp.sum(p * p, axis=1, keepdims=True))
    pn = p / jnp.maximum(nrm, 1e-12)
    a = jnp.dot(pn, w1aT_ref[...], preferred_element_type=jnp.float32) + b1_ref[...]
    b = jnp.dot(pn, w1bT_ref[...], preferred_element_type=jnp.float32)
    t_ref[...] = jnp.concatenate([a, b], axis=-1)


def _edge_prep(feat, wpT, w1aT, w1bT, b1, *, interpret=False):
    BM = 512
    grid = (N // BM,)
    return pl.pallas_call(
        _edge_prep_body,
        grid=grid,
        in_specs=[
            pl.BlockSpec((BM, D), lambda i: (i, 0)),
            pl.BlockSpec((D, SH), lambda i: (0, 0)),
            pl.BlockSpec((SH, SH), lambda i: (0, 0)),
            pl.BlockSpec((SH, SH), lambda i: (0, 0)),
            pl.BlockSpec((1, SH), lambda i: (0, 0)),
        ],
        out_specs=pl.BlockSpec((BM, 2 * SH), lambda i: (i, 0)),
        out_shape=jax.ShapeDtypeStruct((N, 2 * SH), jnp.float32),
        interpret=interpret,
    )(feat, wpT, w1aT, w1bT, b1)


_BE = 12288
_BR = _BE // 128
_DIAG = np.equal(np.arange(_BE)[:, None] % 128, np.arange(128)[None, :]).astype(np.float32)
_GSUM = np.equal(np.arange(_BR)[:, None], np.arange(_BE)[None, :] // 128).astype(np.float32)


def _edge_mlp_body(h_ref, w2r_ref, diag_ref, gsum_ref, b2_ref, w_ref):
    l = _leaky(h_ref[...])
    s = jnp.dot(l, w2r_ref[...], preferred_element_type=jnp.float32)
    ssel = s * diag_ref[...]
    out = jnp.dot(gsum_ref[...], ssel, preferred_element_type=jnp.float32)
    w_ref[...] = jax.nn.sigmoid(out + b2_ref[0])


def _edge_mlp(h, w2rep, b2, *, interpret=False):
    grid = (E // _BE,)
    w2d = pl.pallas_call(
        _edge_mlp_body,
        grid=grid,
        in_specs=[
            pl.BlockSpec((_BE, SH), lambda i: (i, 0)),
            pl.BlockSpec((SH, 128), lambda i: (0, 0)),
            pl.BlockSpec((_BE, 128), lambda i: (0, 0)),
            pl.BlockSpec((_BR, _BE), lambda i: (0, 0)),
            pl.BlockSpec(memory_space=pltpu.SMEM),
        ],
        out_specs=pl.BlockSpec((_BR, 128), lambda i: (i, 0)),
        out_shape=jax.ShapeDtypeStruct((E // 128, 128), jnp.float32),
        interpret=interpret,
    )(h, w2rep, jnp.asarray(_DIAG), jnp.asarray(_GSUM), b2)
    return w2d.reshape(E)


def _q_body(x_ref, w1T_ref, rs_ref, q_ref):
    i = pl.program_id(0)
    rs = rs_ref[pl.ds(i * 512, 512), :]
    d = jnp.where(rs > 0, lax.rsqrt(jnp.where(rs > 0, rs, 1.0)), 0.0)
    q_ref[...] = jnp.dot(x_ref[...], w1T_ref[...], preferred_element_type=jnp.float32) * d


def _q_step(x, w1T, rs2d, *, interpret=False):
    BM = 512
    grid = (N // BM,)
    return pl.pallas_call(
        _q_body,
        grid=grid,
        in_specs=[
            pl.BlockSpec((BM, D), lambda i: (i, 0)),
            pl.BlockSpec((D, D), lambda i: (0, 0)),
            pl.BlockSpec((N, 1), lambda i: (0, 0)),
        ],
        out_specs=pl.BlockSpec((BM, D), lambda i: (i, 0)),
        out_shape=jax.ShapeDtypeStruct((N, D), jnp.float32),
        interpret=interpret,
    )(x, w1T, rs2d)


def _layer_body(adj_ref, q_ref, x_ref, rs_ref, w2aT_ref, w2bT_ref, out_ref, acc_ref):
    m = pl.program_id(0)
    k = pl.program_id(1)
    nk = pl.num_programs(1)
    part = jnp.dot(adj_ref[...], q_ref[...], preferred_element_type=jnp.float32)

    @pl.when(k == 0)
    def _():
        acc_ref[...] = part

    @pl.when(k != 0)
    def _():
        acc_ref[...] += part

    @pl.when(k == nk - 1)
    def _():
        rs = rs_ref[pl.ds(m * 512, 512), :]
        d = jnp.where(rs > 0, lax.rsqrt(jnp.where(rs > 0, rs, 1.0)), 0.0)
        nb = acc_ref[...] * d
        x = x_ref[...]
        u = x + nb
        v = x * nb
        out_ref[...] = _leaky(
            jnp.dot(u, w2aT_ref[...], preferred_element_type=jnp.float32)
            + jnp.dot(v, w2bT_ref[...], preferred_element_type=jnp.float32))


def _layer_step(adj, q, x, rs2d, w2aT, w2bT, *, interpret=False):
    BM, BK = 512, 512
    grid = (N // BM, N // BK)
    return pl.pallas_call(
        _layer_body,
        grid=grid,
        in_specs=[
            pl.BlockSpec((BM, BK), lambda m, k: (m, k)),
            pl.BlockSpec((BK, D), lambda m, k: (k, 0)),
            pl.BlockSpec((BM, D), lambda m, k: (m, 0)),
            pl.BlockSpec((N, 1), lambda m, k: (0, 0)),
            pl.BlockSpec((D, D), lambda m, k: (0, 0)),
            pl.BlockSpec((D, D), lambda m, k: (0, 0)),
        ],
        out_specs=pl.BlockSpec((BM, D), lambda m, k: (m, 0)),
        out_shape=jax.ShapeDtypeStruct((N, D), jnp.float32),
        scratch_shapes=[pltpu.VMEM((BM, D), jnp.float32)],
        interpret=interpret,
    )(adj, q, x, rs2d, w2aT, w2bT)


def _sc_gather(t_tab, src, tgt):
    EW = E // 32
    CH = 128
    NCH = EW // CH

    mesh = plsc.VectorSubcoreMesh(core_axis_name="c", subcore_axis_name="s")

    @functools.partial(
        pl.kernel,
        out_type=jax.ShapeDtypeStruct((E, SH), jnp.float32),
        mesh=mesh,
        scratch_types=[
            pltpu.VMEM((EW,), jnp.int32),
            pltpu.VMEM((EW,), jnp.int32),
            pltpu.VMEM((CH, 2 * SH), jnp.float32),
            pltpu.VMEM((CH, 2 * SH), jnp.float32),
            pltpu.VMEM((CH, 2 * SH), jnp.float32),
            pltpu.VMEM((CH, 2 * SH), jnp.float32),
            pltpu.VMEM((CH, SH), jnp.float32),
            pltpu.VMEM((CH, SH), jnp.float32),
            pltpu.SemaphoreType.DMA,
            pltpu.SemaphoreType.DMA,
        ],
    )
    def k(t_hbm, src_hbm, tgt_hbm, h_hbm,
          idxs, idxt, ra0, ra1, rb0, rb1, h0, h1, semg, semw):
        ra = (ra0, ra1)
        rb = (rb0, rb1)
        hb = (h0, h1)
        wid = lax.axis_index("s") * 2 + lax.axis_index("c")
        base = wid * EW
        pltpu.sync_copy(src_hbm.at[pl.ds(base, EW)], idxs)
        pltpu.sync_copy(tgt_hbm.at[pl.ds(base, EW)], idxt)

        def fire(ci, b):
            pltpu.async_copy(t_hbm.at[idxs.at[pl.ds(ci * CH, CH)]], ra[b], semg)
            pltpu.async_copy(t_hbm.at[idxt.at[pl.ds(ci * CH, CH)]], rb[b], semg)

        def wait_gather(ci, b):
            pltpu.make_async_copy(
                t_hbm.at[idxs.at[pl.ds(ci * CH, CH)]], ra[b], semg).wait()
            pltpu.make_async_copy(
                t_hbm.at[idxt.at[pl.ds(ci * CH, CH)]], rb[b], semg).wait()

        def wdesc(ci, b):
            return pltpu.make_async_copy(
                hb[b], h_hbm.at[pl.ds(base + ci * CH, CH)], semw)

        fire(0, 0)
        for ci in range(NCH):
            b = ci & 1
            if ci + 1 < NCH:
                fire(ci + 1, 1 - b)
            wait_gather(ci, b)
            if ci >= 2:
                wdesc(ci - 2, b).wait()

            def add_edge(e, carry):
                for g in range(SH // 16):
                    hb[b][e, pl.ds(g * 16, 16)] = (
                        ra[b][e, pl.ds(g * 16, 16)]
                        + rb[b][e, pl.ds(SH + g * 16, 16)])
                return carry
            lax.fori_loop(0, CH, add_edge, 0, unroll=4)

            pltpu.async_copy(hb[b], h_hbm.at[pl.ds(base + ci * CH, CH)], semw)
        wdesc(NCH - 2, 0).wait()
        wdesc(NCH - 1, 1).wait()

    return k(t_tab, src, tgt)


_BAND = 384
_NPASS = 4
_PB = _BAND * N
_TSH = _PB // 16
_DUMP = 4096


def _sc_scatter(src, tgt, w):
    mesh = plsc.VectorSubcoreMesh(core_axis_name="c", subcore_axis_name="s")

    @functools.partial(
        pl.kernel,
        out_type=[jax.ShapeDtypeStruct((N, N), jnp.float32),
                  jax.ShapeDtypeStruct((N,), jnp.float32)],
        mesh=mesh,
        scratch_types=[
            pltpu.VMEM_SHARED((_PB + _DUMP,), jnp.float32),
            pltpu.VMEM_SHARED((N,), jnp.float32),
            pltpu.VMEM((8192,), jnp.float32),
            pltpu.VMEM((8, N), jnp.float32),
            pltpu.VMEM((1024,), jnp.int32),
            pltpu.VMEM((1024,), jnp.int32),
            pltpu.VMEM((1024,), jnp.float32),
            pltpu.VMEM((8, 128), jnp.int32),
            pltpu.VMEM((8, 128), jnp.float32),
            pltpu.VMEM((8, 128), jnp.int32),
            pltpu.VMEM((8, 128), jnp.float32),
            pltpu.SemaphoreType.DMA,
        ],
    )
    def k(src_hbm, tgt_hbm, w_hbm, adj_hbm, rs_hbm,
          band_sp, rs_sp, zbuf, bb, cs, ct, cw, oidx, oval, ridx, rval, sem):
        c = lax.axis_index("c")
        s = lax.axis_index("s")
        lane = lax.iota(jnp.int32, 16)

        def zb(i, carry):
            zbuf[pl.ds(i * 16, 16)] = jnp.zeros((16,), jnp.float32)
            return carry
        lax.fori_loop(0, 8192 // 16, zb, 0, unroll=False)

        def one_pass(p, pcarry):
            band = c * _NPASS + p
            lo = band * _BAND

            def zcp(t, carry):
                pltpu.sync_copy(zbuf,
                                band_sp.at[pl.ds(s * _TSH + t * 8192, 8192)])
                return carry
            lax.fori_loop(0, _TSH // 8192, zcp, 0, unroll=False)

            @pl.when(s == 0)
            def _():
                pltpu.sync_copy(zbuf.at[pl.ds(0, _DUMP)], band_sp.at[pl.ds(_PB, _DUMP)])

            @pl.when((p == 0) & (s == 1) & (c == 0))
            def _():
                pltpu.sync_copy(zbuf.at[pl.ds(0, 3072)], rs_sp)

            plsc.subcore_barrier()

            def chunk(ci, carry):
                eb = (s * 6 + ci) * 1024
                pltpu.sync_copy(src_hbm.at[pl.ds(eb, 1024)], cs)
                pltpu.sync_copy(tgt_hbm.at[pl.ds(eb, 1024)], ct)
                pltpu.sync_copy(w_hbm.at[pl.ds(eb, 1024)], cw)
                dump_base = _PB + s * 256
                for g in range(64):
                    r, col = g // 8, (g % 8) * 16
                    s16 = cs[pl.ds(g * 16, 16)]
                    t16 = ct[pl.ds(g * 16, 16)]
                    w16 = cw[pl.ds(g * 16, 16)]
                    m = (s16 >= lo) & (s16 < lo + _BAND)
                    off = (s16 - lo) * N + t16
                    dmp = dump_base + (g % 16) * 16 + lane
                    oidx[r, pl.ds(col, 16)] = jnp.where(m, off, dmp)
                    oval[r, pl.ds(col, 16)] = jnp.where(m, w16, 0.0)
                    ridx[r, pl.ds(col, 16)] = s16
                    rval[r, pl.ds(col, 16)] = w16
                descs = [pltpu.async_copy(oval.at[r], band_sp.at[oidx.at[r]],
                                          sem, add=True) for r in range(8)]

                @pl.when((p == 0) & (c == 0))
                def _():
                    d2 = [pltpu.async_copy(rval.at[r], rs_sp.at[ridx.at[r]],
                                           sem, add=True) for r in range(8)]
                    for d in d2:
                        d.wait()
                for d in descs:
                    d.wait()
                return carry

            lax.fori_loop(0, 6, chunk, 0, unroll=False)
            plsc.subcore_barrier()

            row0 = lo + s * (_TSH // N)

            def ocp(t, carry):
                base_sp = s * _TSH + t * 8 * N
                ds8 = [pltpu.async_copy(band_sp.at[pl.ds(base_sp + r * N, N)],
                                        bb.at[r], sem) for r in range(8)]
                for d in ds8:
                    d.wait()
                pltpu.sync_copy(bb, adj_hbm.at[pl.ds(row0 + t * 8, 8)])
                return carry
            lax.fori_loop(0, _TSH // (8 * N), ocp, 0, unroll=False)

            @pl.when((p == 0) & (c == 0))
            def _():
                pltpu.sync_copy(rs_sp.at[pl.ds(s * 192, 192)],
                                cw.at[pl.ds(0, 192)])
                pltpu.sync_copy(cw.at[pl.ds(0, 192)],
                                rs_hbm.at[pl.ds(s * 192, 192)])
            plsc.subcore_barrier()
            return pcarry

        lax.fori_loop(0, _NPASS, one_pass, 0, unroll=False)

    return k(src, tgt, w)


def kernel(feature_tuple, dia_lens, win_p, win_f, edge_index, ewg_proj_W,
           mlp_W1, mlp_b1, mlp_W2, mlp_b2, l0_W1, l0_W2, l1_W1, l1_W2,
           *, interpret=False):
    feat = feature_tuple.reshape(-1, feature_tuple.shape[-1])
    src, tgt = edge_index[0], edge_index[1]

    wpT = ewg_proj_W.T
    w1aT = mlp_W1[:, :SH].T
    w1bT = mlp_W1[:, SH:].T
    b1 = mlp_b1.reshape(1, SH)
    t_tab = _edge_prep(feat, wpT, w1aT, w1bT, b1, interpret=interpret)

    if interpret:
        h = t_tab[src][:, :SH] + t_tab[tgt][:, SH:]
    else:
        h = _sc_gather(t_tab, src, tgt)

    w = _edge_mlp(h, jnp.tile(mlp_W2.reshape(SH, 1), (1, 128)), mlp_b2,
                  interpret=interpret)

    if interpret:
        adj = jnp.zeros((N, N), jnp.float32).at[src, tgt].add(w)
        rs = jnp.zeros((N,), jnp.float32).at[src].add(w)
    else:
        adj, rs = _sc_scatter(src, tgt, w)
    rs2d = rs.reshape(N, 1)

    x = feat
    for (W1, W2) in [(l0_W1, l0_W2), (l1_W1, l1_W2)]:
        q = _q_step(x, W1.T, rs2d, interpret=interpret)
        x = _layer_step(adj, q, x, rs2d, W2[:, :D].T, W2[:, D:].T,
                        interpret=interpret)

    f0, f1, f2 = jnp.split(x, 3, axis=0)
    return (f0, f1, f2, edge_index)


# speedup vs baseline: 4.0310x; 1.0802x over previous
import functools

import jax
import jax.numpy as jnp
import numpy as np
from jax import lax
from jax.experimental import pallas as pl
from jax.experimental.pallas import tpu as pltpu
from jax.experimental.pallas import tpu_sc as plsc

N = 3072
D = 256
SH = 64
E = 98304


def _leaky(x):
    return jnp.where(x >= 0, x, 0.01 * x)


def _edge_prep_body(feat_ref, wpT_ref, w1aT_ref, w1bT_ref, b1_ref, t_ref):
    p = jnp.dot(feat_ref[...], wpT_ref[...], preferred_element_type=jnp.float32)
    nrm = jnp.sqrt(jnp.sum(p * p, axis=1, keepdims=True))
    pn = p / jnp.maximum(nrm, 1e-12)
    a = jnp.dot(pn, w1aT_ref[...], preferred_element_type=jnp.float32) + b1_ref[...]
    b = jnp.dot(pn, w1bT_ref[...], preferred_element_type=jnp.float32)
    t_ref[...] = jnp.concatenate([a, b], axis=-1)


def _edge_prep(feat, wpT, w1aT, w1bT, b1, *, interpret=False):
    BM = 512
    grid = (N // BM,)
    return pl.pallas_call(
        _edge_prep_body,
        grid=grid,
        in_specs=[
            pl.BlockSpec((BM, D), lambda i: (i, 0)),
            pl.BlockSpec((D, SH), lambda i: (0, 0)),
            pl.BlockSpec((SH, SH), lambda i: (0, 0)),
            pl.BlockSpec((SH, SH), lambda i: (0, 0)),
            pl.BlockSpec((1, SH), lambda i: (0, 0)),
        ],
        out_specs=pl.BlockSpec((BM, 2 * SH), lambda i: (i, 0)),
        out_shape=jax.ShapeDtypeStruct((N, 2 * SH), jnp.float32),
        interpret=interpret,
    )(feat, wpT, w1aT, w1bT, b1)


_BE = 12288
_BR = _BE // 128
_DIAG = np.equal(np.arange(_BE)[:, None] % 128, np.arange(128)[None, :]).astype(np.float32)
_GSUM = np.equal(np.arange(_BR)[:, None], np.arange(_BE)[None, :] // 128).astype(np.float32)


def _edge_mlp_body(h_ref, w2r_ref, diag_ref, gsum_ref, b2_ref, w_ref):
    l = _leaky(h_ref[...])
    s = jnp.dot(l, w2r_ref[...], preferred_element_type=jnp.float32)
    ssel = s * diag_ref[...]
    out = jnp.dot(gsum_ref[...], ssel, preferred_element_type=jnp.float32)
    w_ref[...] = jax.nn.sigmoid(out + b2_ref[0])


def _edge_mlp(h, w2rep, b2, *, interpret=False):
    grid = (E // _BE,)
    w2d = pl.pallas_call(
        _edge_mlp_body,
        grid=grid,
        in_specs=[
            pl.BlockSpec((_BE, SH), lambda i: (i, 0)),
            pl.BlockSpec((SH, 128), lambda i: (0, 0)),
            pl.BlockSpec((_BE, 128), lambda i: (0, 0)),
            pl.BlockSpec((_BR, _BE), lambda i: (0, 0)),
            pl.BlockSpec(memory_space=pltpu.SMEM),
        ],
        out_specs=pl.BlockSpec((_BR, 128), lambda i: (i, 0)),
        out_shape=jax.ShapeDtypeStruct((E // 128, 128), jnp.float32),
        interpret=interpret,
    )(h, w2rep, jnp.asarray(_DIAG), jnp.asarray(_GSUM), b2)
    return w2d.reshape(E)


def _gcn_body(adj_ref, feat_ref, rs_ref, w1T_ref, w2aT_ref, w2bT_ref,
              out_ref, q_all, x_next, acc_ref):
    l = pl.program_id(0)
    m = pl.program_id(1)
    k = pl.program_id(2)
    nk = pl.num_programs(2)

    def dvec(i):
        rs = rs_ref[pl.ds(i * 512, 512), :]
        return jnp.where(rs > 0, lax.rsqrt(jnp.where(rs > 0, rs, 1.0)), 0.0)

    def xblk(i):
        return jnp.where(l == 0, feat_ref[pl.ds(i * 512, 512), :],
                         x_next[pl.ds(i * 512, 512), :])

    @pl.when(m == 0)
    def _():
        q_all[pl.ds(k * 512, 512), :] = jnp.dot(
            xblk(k), w1T_ref[0], preferred_element_type=jnp.float32) * dvec(k)

    part = jnp.dot(adj_ref[...], q_all[pl.ds(k * 512, 512), :],
                   preferred_element_type=jnp.float32)

    @pl.when(k == 0)
    def _():
        acc_ref[...] = part

    @pl.when(k != 0)
    def _():
        acc_ref[...] += part

    @pl.when(k == nk - 1)
    def _():
        nb = acc_ref[...] * dvec(m)
        x = xblk(m)
        u = x + nb
        v = x * nb
        y = _leaky(
            jnp.dot(u, w2aT_ref[0], preferred_element_type=jnp.float32)
            + jnp.dot(v, w2bT_ref[0], preferred_element_type=jnp.float32))

        @pl.when(l == 0)
        def _():
            x_next[pl.ds(m * 512, 512), :] = y

        @pl.when(l == 1)
        def _():
            out_ref[...] = y


def _gcn_layers(adj, feat, rs2d, w1Ts, w2aTs, w2bTs, *, interpret=False):
    BM, BK = 512, 512
    grid = (2, N // BM, N // BK)
    return pl.pallas_call(
        _gcn_body,
        grid=grid,
        in_specs=[
            pl.BlockSpec((BM, BK), lambda l, m, k: (m, k)),
            pl.BlockSpec((N, D), lambda l, m, k: (0, 0)),
            pl.BlockSpec((N, 1), lambda l, m, k: (0, 0)),
            pl.BlockSpec((1, D, D), lambda l, m, k: (l, 0, 0)),
            pl.BlockSpec((1, D, D), lambda l, m, k: (l, 0, 0)),
            pl.BlockSpec((1, D, D), lambda l, m, k: (l, 0, 0)),
        ],
        out_specs=pl.BlockSpec((BM, D), lambda l, m, k: (m, 0)),
        out_shape=jax.ShapeDtypeStruct((N, D), jnp.float32),
        scratch_shapes=[
            pltpu.VMEM((N, D), jnp.float32),
            pltpu.VMEM((N, D), jnp.float32),
            pltpu.VMEM((BM, D), jnp.float32),
        ],
        interpret=interpret,
    )(adj, feat, rs2d, w1Ts, w2aTs, w2bTs)


def _sc_gather(t_tab, src, tgt):
    EW = E // 32
    CH = 128
    NCH = EW // CH

    mesh = plsc.VectorSubcoreMesh(core_axis_name="c", subcore_axis_name="s")

    @functools.partial(
        pl.kernel,
        out_type=jax.ShapeDtypeStruct((E, SH), jnp.float32),
        mesh=mesh,
        scratch_types=[
            pltpu.VMEM((EW,), jnp.int32),
            pltpu.VMEM((EW,), jnp.int32),
            pltpu.VMEM((CH, 2 * SH), jnp.float32),
            pltpu.VMEM((CH, 2 * SH), jnp.float32),
            pltpu.VMEM((CH, 2 * SH), jnp.float32),
            pltpu.VMEM((CH, 2 * SH), jnp.float32),
            pltpu.VMEM((CH, SH), jnp.float32),
            pltpu.VMEM((CH, SH), jnp.float32),
            pltpu.SemaphoreType.DMA,
            pltpu.SemaphoreType.DMA,
        ],
    )
    def k(t_hbm, src_hbm, tgt_hbm, h_hbm,
          idxs, idxt, ra0, ra1, rb0, rb1, h0, h1, semg, semw):
        ra = (ra0, ra1)
        rb = (rb0, rb1)
        hb = (h0, h1)
        wid = lax.axis_index("s") * 2 + lax.axis_index("c")
        base = wid * EW
        pltpu.sync_copy(src_hbm.at[pl.ds(base, EW)], idxs)
        pltpu.sync_copy(tgt_hbm.at[pl.ds(base, EW)], idxt)

        def fire(ci, b):
            pltpu.async_copy(t_hbm.at[idxs.at[pl.ds(ci * CH, CH)]], ra[b], semg)
            pltpu.async_copy(t_hbm.at[idxt.at[pl.ds(ci * CH, CH)]], rb[b], semg)

        def wait_gather(ci, b):
            pltpu.make_async_copy(
                t_hbm.at[idxs.at[pl.ds(ci * CH, CH)]], ra[b], semg).wait()
            pltpu.make_async_copy(
                t_hbm.at[idxt.at[pl.ds(ci * CH, CH)]], rb[b], semg).wait()

        def wdesc(ci, b):
            return pltpu.make_async_copy(
                hb[b], h_hbm.at[pl.ds(base + ci * CH, CH)], semw)

        fire(0, 0)
        for ci in range(NCH):
            b = ci & 1
            if ci + 1 < NCH:
                fire(ci + 1, 1 - b)
            wait_gather(ci, b)
            if ci >= 2:
                wdesc(ci - 2, b).wait()

            def add_edge(e, carry):
                for g in range(SH // 16):
                    hb[b][e, pl.ds(g * 16, 16)] = (
                        ra[b][e, pl.ds(g * 16, 16)]
                        + rb[b][e, pl.ds(SH + g * 16, 16)])
                return carry
            lax.fori_loop(0, CH, add_edge, 0, unroll=4)

            pltpu.async_copy(hb[b], h_hbm.at[pl.ds(base + ci * CH, CH)], semw)
        wdesc(NCH - 2, 0).wait()
        wdesc(NCH - 1, 1).wait()

    return k(t_tab, src, tgt)


_BAND = 384
_NPASS = 4
_PB = _BAND * N
_TSH = _PB // 16
_DUMP = 4096


def _sc_scatter(src, tgt, w):
    mesh = plsc.VectorSubcoreMesh(core_axis_name="c", subcore_axis_name="s")

    @functools.partial(
        pl.kernel,
        out_type=[jax.ShapeDtypeStruct((N, N), jnp.float32),
                  jax.ShapeDtypeStruct((N,), jnp.float32)],
        mesh=mesh,
        scratch_types=[
            pltpu.VMEM_SHARED((_PB + _DUMP,), jnp.float32),
            pltpu.VMEM_SHARED((N,), jnp.float32),
            pltpu.VMEM((8192,), jnp.float32),
            pltpu.VMEM((8, N), jnp.float32),
            pltpu.VMEM((1024,), jnp.int32),
            pltpu.VMEM((1024,), jnp.int32),
            pltpu.VMEM((1024,), jnp.float32),
            pltpu.VMEM((8, 128), jnp.int32),
            pltpu.VMEM((8, 128), jnp.float32),
            pltpu.VMEM((8, 128), jnp.int32),
            pltpu.VMEM((8, 128), jnp.float32),
            pltpu.SemaphoreType.DMA,
        ],
    )
    def k(src_hbm, tgt_hbm, w_hbm, adj_hbm, rs_hbm,
          band_sp, rs_sp, zbuf, bb, cs, ct, cw, oidx, oval, ridx, rval, sem):
        c = lax.axis_index("c")
        s = lax.axis_index("s")
        lane = lax.iota(jnp.int32, 16)

        def zb(i, carry):
            zbuf[pl.ds(i * 16, 16)] = jnp.zeros((16,), jnp.float32)
            return carry
        lax.fori_loop(0, 8192 // 16, zb, 0, unroll=False)

        def one_pass(p, pcarry):
            band = c * _NPASS + p
            lo = band * _BAND

            def zcp(t, carry):
                pltpu.sync_copy(zbuf,
                                band_sp.at[pl.ds(s * _TSH + t * 8192, 8192)])
                return carry
            lax.fori_loop(0, _TSH // 8192, zcp, 0, unroll=False)

            @pl.when(s == 0)
            def _():
                pltpu.sync_copy(zbuf.at[pl.ds(0, _DUMP)], band_sp.at[pl.ds(_PB, _DUMP)])

            @pl.when((p == 0) & (s == 1) & (c == 0))
            def _():
                pltpu.sync_copy(zbuf.at[pl.ds(0, 3072)], rs_sp)

            plsc.subcore_barrier()

            def chunk(ci, carry):
                eb = (s * 6 + ci) * 1024
                pltpu.sync_copy(src_hbm.at[pl.ds(eb, 1024)], cs)
                pltpu.sync_copy(tgt_hbm.at[pl.ds(eb, 1024)], ct)
                pltpu.sync_copy(w_hbm.at[pl.ds(eb, 1024)], cw)
                dump_base = _PB + s * 256
                for g in range(64):
                    r, col = g // 8, (g % 8) * 16
                    s16 = cs[pl.ds(g * 16, 16)]
                    t16 = ct[pl.ds(g * 16, 16)]
                    w16 = cw[pl.ds(g * 16, 16)]
                    m = (s16 >= lo) & (s16 < lo + _BAND)
                    off = (s16 - lo) * N + t16
                    dmp = dump_base + (g % 16) * 16 + lane
                    oidx[r, pl.ds(col, 16)] = jnp.where(m, off, dmp)
                    oval[r, pl.ds(col, 16)] = jnp.where(m, w16, 0.0)
                    ridx[r, pl.ds(col, 16)] = s16
                    rval[r, pl.ds(col, 16)] = w16
                descs = [pltpu.async_copy(oval.at[r], band_sp.at[oidx.at[r]],
                                          sem, add=True) for r in range(8)]

                @pl.when((p == 0) & (c == 0))
                def _():
                    d2 = [pltpu.async_copy(rval.at[r], rs_sp.at[ridx.at[r]],
                                           sem, add=True) for r in range(8)]
                    for d in d2:
                        d.wait()
                for d in descs:
                    d.wait()
                return carry

            lax.fori_loop(0, 6, chunk, 0, unroll=False)
            plsc.subcore_barrier()

            row0 = lo + s * (_TSH // N)

            def ocp(t, carry):
                base_sp = s * _TSH + t * 8 * N
                ds8 = [pltpu.async_copy(band_sp.at[pl.ds(base_sp + r * N, N)],
                                        bb.at[r], sem) for r in range(8)]
                for d in ds8:
                    d.wait()
                pltpu.sync_copy(bb, adj_hbm.at[pl.ds(row0 + t * 8, 8)])
                return carry
            lax.fori_loop(0, _TSH // (8 * N), ocp, 0, unroll=False)

            @pl.when((p == 0) & (c == 0))
            def _():
                pltpu.sync_copy(rs_sp.at[pl.ds(s * 192, 192)],
                                cw.at[pl.ds(0, 192)])
                pltpu.sync_copy(cw.at[pl.ds(0, 192)],
                                rs_hbm.at[pl.ds(s * 192, 192)])
            plsc.subcore_barrier()
            return pcarry

        lax.fori_loop(0, _NPASS, one_pass, 0, unroll=False)

    return k(src, tgt, w)


def kernel(feature_tuple, dia_lens, win_p, win_f, edge_index, ewg_proj_W,
           mlp_W1, mlp_b1, mlp_W2, mlp_b2, l0_W1, l0_W2, l1_W1, l1_W2,
           *, interpret=False):
    feat = feature_tuple.reshape(-1, feature_tuple.shape[-1])
    src, tgt = edge_index[0], edge_index[1]

    wpT = ewg_proj_W.T
    w1aT = mlp_W1[:, :SH].T
    w1bT = mlp_W1[:, SH:].T
    b1 = mlp_b1.reshape(1, SH)
    t_tab = _edge_prep(feat, wpT, w1aT, w1bT, b1, interpret=interpret)

    if interpret:
        h = t_tab[src][:, :SH] + t_tab[tgt][:, SH:]
    else:
        h = _sc_gather(t_tab, src, tgt)

    w = _edge_mlp(h, jnp.tile(mlp_W2.reshape(SH, 1), (1, 128)), mlp_b2,
                  interpret=interpret)

    if interpret:
        adj = jnp.zeros((N, N), jnp.float32).at[src, tgt].add(w)
        rs = jnp.zeros((N,), jnp.float32).at[src].add(w)
    else:
        adj, rs = _sc_scatter(src, tgt, w)
    rs2d = rs.reshape(N, 1)

    w1Ts = jnp.stack([l0_W1.T, l1_W1.T])
    w2aTs = jnp.stack([l0_W2[:, :D].T, l1_W2[:, :D].T])
    w2bTs = jnp.stack([l0_W2[:, D:].T, l1_W2[:, D:].T])
    x = _gcn_layers(adj, feat, rs2d, w1Ts, w2aTs, w2bTs, interpret=interpret)

    f0, f1, f2 = jnp.split(x, 3, axis=0)
    return (f0, f1, f2, edge_index)


# speedup vs baseline: 4.3949x; 1.0903x over previous
import functools

import jax
import jax.numpy as jnp
import numpy as np
from jax import lax
from jax.experimental import pallas as pl
from jax.experimental.pallas import tpu as pltpu
from jax.experimental.pallas import tpu_sc as plsc

N = 3072
D = 256
SH = 64
E = 98304


def _leaky(x):
    return jnp.where(x >= 0, x, 0.01 * x)


def _edge_prep_body(feat_ref, wpT_ref, w1aT_ref, w1bT_ref, b1_ref, t_ref):
    p = jnp.dot(feat_ref[...], wpT_ref[...], preferred_element_type=jnp.float32)
    nrm = jnp.sqrt(jnp.sum(p * p, axis=1, keepdims=True))
    pn = p / jnp.maximum(nrm, 1e-12)
    a = jnp.dot(pn, w1aT_ref[...], preferred_element_type=jnp.float32) + b1_ref[...]
    b = jnp.dot(pn, w1bT_ref[...], preferred_element_type=jnp.float32)
    t_ref[...] = jnp.concatenate([a, b], axis=-1)


def _edge_prep(feat, wpT, w1aT, w1bT, b1, *, interpret=False):
    BM = 512
    grid = (N // BM,)
    return pl.pallas_call(
        _edge_prep_body,
        grid=grid,
        in_specs=[
            pl.BlockSpec((BM, D), lambda i: (i, 0)),
            pl.BlockSpec((D, SH), lambda i: (0, 0)),
            pl.BlockSpec((SH, SH), lambda i: (0, 0)),
            pl.BlockSpec((SH, SH), lambda i: (0, 0)),
            pl.BlockSpec((1, SH), lambda i: (0, 0)),
        ],
        out_specs=pl.BlockSpec((BM, 2 * SH), lambda i: (i, 0)),
        out_shape=jax.ShapeDtypeStruct((N, 2 * SH), jnp.float32),
        interpret=interpret,
    )(feat, wpT, w1aT, w1bT, b1)


_BE = 12288
_BR = _BE // 128
_DIAG = np.equal(np.arange(_BE)[:, None] % 128, np.arange(128)[None, :]).astype(np.float32)
_GSUM = np.equal(np.arange(_BR)[:, None], np.arange(_BE)[None, :] // 128).astype(np.float32)


def _edge_mlp_body(h_ref, w2r_ref, diag_ref, gsum_ref, b2_ref, w_ref):
    l = _leaky(h_ref[...])
    s = jnp.dot(l, w2r_ref[...], preferred_element_type=jnp.float32)
    ssel = s * diag_ref[...]
    out = jnp.dot(gsum_ref[...], ssel, preferred_element_type=jnp.float32)
    w_ref[...] = jax.nn.sigmoid(out + b2_ref[0])


def _edge_mlp(h, w2rep, b2, *, interpret=False):
    grid = (E // _BE,)
    w2d = pl.pallas_call(
        _edge_mlp_body,
        grid=grid,
        in_specs=[
            pl.BlockSpec((_BE, SH), lambda i: (i, 0)),
            pl.BlockSpec((SH, 128), lambda i: (0, 0)),
            pl.BlockSpec((_BE, 128), lambda i: (0, 0)),
            pl.BlockSpec((_BR, _BE), lambda i: (0, 0)),
            pl.BlockSpec(memory_space=pltpu.SMEM),
        ],
        out_specs=pl.BlockSpec((_BR, 128), lambda i: (i, 0)),
        out_shape=jax.ShapeDtypeStruct((E // 128, 128), jnp.float32),
        interpret=interpret,
    )(h, w2rep, jnp.asarray(_DIAG), jnp.asarray(_GSUM), b2)
    return w2d.reshape(E)


def _gcn_body(adj_ref, feat_ref, rs_ref, w1T_ref, w2aT_ref, w2bT_ref,
              out_ref, q_all, x_next, acc_ref):
    l = pl.program_id(0)
    m = pl.program_id(1)
    k = pl.program_id(2)
    nk = pl.num_programs(2)

    def dvec(i):
        rs = rs_ref[pl.ds(i * 512, 512), :]
        return jnp.where(rs > 0, lax.rsqrt(jnp.where(rs > 0, rs, 1.0)), 0.0)

    def xblk(i):
        return jnp.where(l == 0, feat_ref[pl.ds(i * 512, 512), :],
                         x_next[pl.ds(i * 512, 512), :])

    @pl.when(m == 0)
    def _():
        q_all[pl.ds(k * 512, 512), :] = jnp.dot(
            xblk(k), w1T_ref[0], preferred_element_type=jnp.float32) * dvec(k)

    part = jnp.dot(adj_ref[...], q_all[pl.ds(k * 512, 512), :],
                   preferred_element_type=jnp.float32)

    @pl.when(k == 0)
    def _():
        acc_ref[...] = part

    @pl.when(k != 0)
    def _():
        acc_ref[...] += part

    @pl.when(k == nk - 1)
    def _():
        nb = acc_ref[...] * dvec(m)
        x = xblk(m)
        u = x + nb
        v = x * nb
        y = _leaky(
            jnp.dot(u, w2aT_ref[0], preferred_element_type=jnp.float32)
            + jnp.dot(v, w2bT_ref[0], preferred_element_type=jnp.float32))

        @pl.when(l == 0)
        def _():
            x_next[pl.ds(m * 512, 512), :] = y

        @pl.when(l == 1)
        def _():
            out_ref[...] = y


def _gcn_layers(adj, feat, rs2d, w1Ts, w2aTs, w2bTs, *, interpret=False):
    BM, BK = 512, 512
    grid = (2, N // BM, N // BK)
    return pl.pallas_call(
        _gcn_body,
        grid=grid,
        in_specs=[
            pl.BlockSpec((BM, BK), lambda l, m, k: (m, k)),
            pl.BlockSpec((N, D), lambda l, m, k: (0, 0)),
            pl.BlockSpec((N, 1), lambda l, m, k: (0, 0)),
            pl.BlockSpec((1, D, D), lambda l, m, k: (l, 0, 0)),
            pl.BlockSpec((1, D, D), lambda l, m, k: (l, 0, 0)),
            pl.BlockSpec((1, D, D), lambda l, m, k: (l, 0, 0)),
        ],
        out_specs=pl.BlockSpec((BM, D), lambda l, m, k: (m, 0)),
        out_shape=jax.ShapeDtypeStruct((N, D), jnp.float32),
        scratch_shapes=[
            pltpu.VMEM((N, D), jnp.float32),
            pltpu.VMEM((N, D), jnp.float32),
            pltpu.VMEM((BM, D), jnp.float32),
        ],
        interpret=interpret,
    )(adj, feat, rs2d, w1Ts, w2aTs, w2bTs)


def _sc_gather(t_tab, src, tgt):
    EW = E // 32
    CH = 128
    NCH = EW // CH

    mesh = plsc.VectorSubcoreMesh(core_axis_name="c", subcore_axis_name="s")

    @functools.partial(
        pl.kernel,
        out_type=jax.ShapeDtypeStruct((E, SH), jnp.float32),
        mesh=mesh,
        scratch_types=[
            pltpu.VMEM((EW,), jnp.int32),
            pltpu.VMEM((EW,), jnp.int32),
            pltpu.VMEM((CH, 2 * SH), jnp.float32),
            pltpu.VMEM((CH, 2 * SH), jnp.float32),
            pltpu.VMEM((CH, 2 * SH), jnp.float32),
            pltpu.VMEM((CH, 2 * SH), jnp.float32),
            pltpu.VMEM((CH, SH), jnp.float32),
            pltpu.VMEM((CH, SH), jnp.float32),
            pltpu.SemaphoreType.DMA,
            pltpu.SemaphoreType.DMA,
        ],
    )
    def k(t_hbm, src_hbm, tgt_hbm, h_hbm,
          idxs, idxt, ra0, ra1, rb0, rb1, h0, h1, semg, semw):
        ra = (ra0, ra1)
        rb = (rb0, rb1)
        hb = (h0, h1)
        wid = lax.axis_index("s") * 2 + lax.axis_index("c")
        base = wid * EW
        pltpu.sync_copy(src_hbm.at[pl.ds(base, EW)], idxs)
        pltpu.sync_copy(tgt_hbm.at[pl.ds(base, EW)], idxt)

        def fire(ci, b):
            pltpu.async_copy(t_hbm.at[idxs.at[pl.ds(ci * CH, CH)]], ra[b], semg)
            pltpu.async_copy(t_hbm.at[idxt.at[pl.ds(ci * CH, CH)]], rb[b], semg)

        def wait_gather(ci, b):
            pltpu.make_async_copy(
                t_hbm.at[idxs.at[pl.ds(ci * CH, CH)]], ra[b], semg).wait()
            pltpu.make_async_copy(
                t_hbm.at[idxt.at[pl.ds(ci * CH, CH)]], rb[b], semg).wait()

        def wdesc(ci, b):
            return pltpu.make_async_copy(
                hb[b], h_hbm.at[pl.ds(base + ci * CH, CH)], semw)

        fire(0, 0)
        for ci in range(NCH):
            b = ci & 1
            if ci + 1 < NCH:
                fire(ci + 1, 1 - b)
            wait_gather(ci, b)
            if ci >= 2:
                wdesc(ci - 2, b).wait()

            def add_edge(e, carry):
                for g in range(SH // 16):
                    hb[b][e, pl.ds(g * 16, 16)] = (
                        ra[b][e, pl.ds(g * 16, 16)]
                        + rb[b][e, pl.ds(SH + g * 16, 16)])
                return carry
            lax.fori_loop(0, CH, add_edge, 0, unroll=4)

            pltpu.async_copy(hb[b], h_hbm.at[pl.ds(base + ci * CH, CH)], semw)
        wdesc(NCH - 2, 0).wait()
        wdesc(NCH - 1, 1).wait()

    return k(t_tab, src, tgt)


_BAND = 384
_NPASS = 4
_PB = _BAND * N
_TSH = _PB // 16
_DUMP = 4096


def _sc_scatter(src, tgt, w):
    mesh = plsc.VectorSubcoreMesh(core_axis_name="c", subcore_axis_name="s")

    @functools.partial(
        pl.kernel,
        out_type=[jax.ShapeDtypeStruct((N, N), jnp.float32),
                  jax.ShapeDtypeStruct((N,), jnp.float32)],
        mesh=mesh,
        scratch_types=[
            pltpu.VMEM_SHARED((_PB + _DUMP,), jnp.float32),
            pltpu.VMEM_SHARED((N,), jnp.float32),
            pltpu.VMEM((8192,), jnp.float32),
            pltpu.VMEM((8, N), jnp.float32),
            pltpu.VMEM((1024,), jnp.int32),
            pltpu.VMEM((1024,), jnp.int32),
            pltpu.VMEM((1024,), jnp.float32),
            pltpu.VMEM((8, 128), jnp.int32),
            pltpu.VMEM((8, 128), jnp.float32),
            pltpu.VMEM((8, 128), jnp.int32),
            pltpu.VMEM((8, 128), jnp.float32),
            pltpu.SemaphoreType.DMA,
        ],
    )
    def k(src_hbm, tgt_hbm, w_hbm, adj_hbm, rs_hbm,
          band_sp, rs_sp, zbuf, bb, cs, ct, cw, oidx, oval, ridx, rval, sem):
        c = lax.axis_index("c")
        s = lax.axis_index("s")
        lane = lax.iota(jnp.int32, 16)

        def zb(i, carry):
            zbuf[pl.ds(i * 16, 16)] = jnp.zeros((16,), jnp.float32)
            return carry
        lax.fori_loop(0, 8192 // 16, zb, 0, unroll=False)

        def one_pass(p, pcarry):
            band = c * _NPASS + p
            lo = band * _BAND

            zds = [pltpu.async_copy(zbuf,
                                    band_sp.at[pl.ds(s * _TSH + t * 8192, 8192)],
                                    sem)
                   for t in range(_TSH // 8192)]

            @pl.when(s == 0)
            def _():
                pltpu.sync_copy(zbuf.at[pl.ds(0, _DUMP)], band_sp.at[pl.ds(_PB, _DUMP)])
            for d in zds:
                d.wait()

            @pl.when((p == 0) & (s == 1) & (c == 0))
            def _():
                pltpu.sync_copy(zbuf.at[pl.ds(0, 3072)], rs_sp)

            plsc.subcore_barrier()

            def chunk(ci, carry):
                eb = (s * 6 + ci) * 1024
                dls = [pltpu.async_copy(src_hbm.at[pl.ds(eb, 1024)], cs, sem),
                       pltpu.async_copy(tgt_hbm.at[pl.ds(eb, 1024)], ct, sem),
                       pltpu.async_copy(w_hbm.at[pl.ds(eb, 1024)], cw, sem)]
                for d in dls:
                    d.wait()
                dump_base = _PB + s * 256
                for g in range(64):
                    r, col = g // 8, (g % 8) * 16
                    s16 = cs[pl.ds(g * 16, 16)]
                    t16 = ct[pl.ds(g * 16, 16)]
                    w16 = cw[pl.ds(g * 16, 16)]
                    m = (s16 >= lo) & (s16 < lo + _BAND)
                    off = (s16 - lo) * N + t16
                    dmp = dump_base + (g % 16) * 16 + lane
                    oidx[r, pl.ds(col, 16)] = jnp.where(m, off, dmp)
                    oval[r, pl.ds(col, 16)] = jnp.where(m, w16, 0.0)
                    ridx[r, pl.ds(col, 16)] = s16
                    rval[r, pl.ds(col, 16)] = w16
                descs = [pltpu.async_copy(oval.at[r], band_sp.at[oidx.at[r]],
                                          sem, add=True) for r in range(8)]

                @pl.when((p == 0) & (c == 0))
                def _():
                    d2 = [pltpu.async_copy(rval.at[r], rs_sp.at[ridx.at[r]],
                                           sem, add=True) for r in range(8)]
                    for d in d2:
                        d.wait()
                for d in descs:
                    d.wait()
                return carry

            lax.fori_loop(0, 6, chunk, 0, unroll=False)
            plsc.subcore_barrier()

            row0 = lo + s * (_TSH // N)

            def ocp(t, carry):
                base_sp = s * _TSH + t * 8 * N
                ds8 = [pltpu.async_copy(band_sp.at[pl.ds(base_sp + r * N, N)],
                                        bb.at[r], sem) for r in range(8)]
                for d in ds8:
                    d.wait()
                pltpu.sync_copy(bb, adj_hbm.at[pl.ds(row0 + t * 8, 8)])
                return carry
            lax.fori_loop(0, _TSH // (8 * N), ocp, 0, unroll=False)

            @pl.when((p == 0) & (c == 0))
            def _():
                pltpu.sync_copy(rs_sp.at[pl.ds(s * 192, 192)],
                                cw.at[pl.ds(0, 192)])
                pltpu.sync_copy(cw.at[pl.ds(0, 192)],
                                rs_hbm.at[pl.ds(s * 192, 192)])
            plsc.subcore_barrier()
            return pcarry

        lax.fori_loop(0, _NPASS, one_pass, 0, unroll=False)

    return k(src, tgt, w)


def kernel(feature_tuple, dia_lens, win_p, win_f, edge_index, ewg_proj_W,
           mlp_W1, mlp_b1, mlp_W2, mlp_b2, l0_W1, l0_W2, l1_W1, l1_W2,
           *, interpret=False):
    feat = feature_tuple.reshape(-1, feature_tuple.shape[-1])
    src, tgt = edge_index[0], edge_index[1]

    wpT = ewg_proj_W.T
    w1aT = mlp_W1[:, :SH].T
    w1bT = mlp_W1[:, SH:].T
    b1 = mlp_b1.reshape(1, SH)
    t_tab = _edge_prep(feat, wpT, w1aT, w1bT, b1, interpret=interpret)

    if interpret:
        h = t_tab[src][:, :SH] + t_tab[tgt][:, SH:]
    else:
        h = _sc_gather(t_tab, src, tgt)

    w = _edge_mlp(h, jnp.tile(mlp_W2.reshape(SH, 1), (1, 128)), mlp_b2,
                  interpret=interpret)

    if interpret:
        adj = jnp.zeros((N, N), jnp.float32).at[src, tgt].add(w)
        rs = jnp.zeros((N,), jnp.float32).at[src].add(w)
    else:
        adj, rs = _sc_scatter(src, tgt, w)
    rs2d = rs.reshape(N, 1)

    w1Ts = jnp.stack([l0_W1.T, l1_W1.T])
    w2aTs = jnp.stack([l0_W2[:, :D].T, l1_W2[:, :D].T])
    w2bTs = jnp.stack([l0_W2[:, D:].T, l1_W2[:, D:].T])
    x = _gcn_layers(adj, feat, rs2d, w1Ts, w2aTs, w2bTs, interpret=interpret)

    f0, f1, f2 = jnp.split(x, 3, axis=0)
    return (f0, f1, f2, edge_index)
